# baseline (speedup 1.0000x reference)
import functools

import jax
import jax.numpy as jnp
from jax import lax
from jax.experimental import pallas as pl
from jax.experimental.pallas import tpu as pltpu
from jax.experimental.pallas import tpu_sc as plsc

N = 10000
D = 128
N_CLS = 64
NP = 10240
RPT = NP // 32
W = 384
WU = 320
E_REAL = 170000
EP = 172032
EPT = EP // 32


def _lin_kernel(x_ref, w_ref, y_ref, ns_ref):
    y = jnp.dot(x_ref[...], w_ref[...].T, preferred_element_type=jnp.float32)
    y_ref[...] = y
    ns = jnp.sqrt(jnp.sum(y * y, axis=-1, keepdims=True))
    ns_ref[...] = jnp.maximum(ns, 1e-6)


def _linear(x, Wm):
    return pl.pallas_call(
        _lin_kernel,
        grid=(NP // 512,),
        in_specs=[
            pl.BlockSpec((512, D), lambda i: (i, 0)),
            pl.BlockSpec((D, D), lambda i: (0, 0)),
        ],
        out_specs=[
            pl.BlockSpec((512, D), lambda i: (i, 0)),
            pl.BlockSpec((512, 1), lambda i: (i, 0)),
        ],
        out_shape=[
            jax.ShapeDtypeStruct((NP, D), jnp.float32),
            jax.ShapeDtypeStruct((NP, 1), jnp.float32),
        ],
    )(x, Wm)


def _head_kernel(h_ref, w_ref, b_ref, o_ref):
    logits = jnp.dot(h_ref[...], w_ref[...].T,
                     preferred_element_type=jnp.float32) + b_ref[...]
    m = jnp.max(logits, axis=-1, keepdims=True)
    z = logits - m
    lse = jnp.log(jnp.sum(jnp.exp(z), axis=-1, keepdims=True))
    o_ref[...] = z - lse


def _head(h, W_out, b_out):
    out = pl.pallas_call(
        _head_kernel,
        grid=(NP // 512,),
        in_specs=[
            pl.BlockSpec((512, D), lambda i: (i, 0)),
            pl.BlockSpec((N_CLS, D), lambda i: (0, 0)),
            pl.BlockSpec((1, N_CLS), lambda i: (0, 0)),
        ],
        out_specs=pl.BlockSpec((512, N_CLS), lambda i: (i, 0)),
        out_shape=jax.ShapeDtypeStruct((NP, N_CLS), jnp.float32),
    )(h, W_out, b_out.reshape(1, N_CLS))
    return out[:N]


CE = 8192
_STAGE = 5
_MESH = plsc.VectorSubcoreMesh(core_axis_name="c", subcore_axis_name="s")

_DNUMS = lax.GatherDimensionNumbers(
    offset_dims=(), collapsed_slice_dims=(0,), start_index_map=(0,))


def _take16(v, idx):
    return lax.gather(v, idx[:, None], _DNUMS, (1,),
                      mode=lax.GatherScatterMode.PROMISE_IN_BOUNDS)


def _prefix16(m, iota16):
    zero = jnp.zeros((16,), jnp.int32)
    v = jnp.where(m, jnp.full((16,), 1, jnp.int32), zero)
    for k in (1, 2, 4, 8):
        g = _take16(v, jnp.maximum(iota16 - k, 0))
        v = v + jnp.where(iota16 >= k, g, zero)
    return v - 1, v[15]


def _popcount16(v):
    c55 = jnp.full((16,), 0x55555555, jnp.int32)
    c33 = jnp.full((16,), 0x33333333, jnp.int32)
    c0f = jnp.full((16,), 0x0F0F0F0F, jnp.int32)
    c01 = jnp.full((16,), 0x01010101, jnp.int32)
    v = v - (lax.shift_right_logical(v, 1) & c55)
    v = (v & c33) + (lax.shift_right_logical(v, 2) & c33)
    v = (v + lax.shift_right_logical(v, 4)) & c0f
    return lax.shift_right_logical(v * c01, 24)


AE = 512


@functools.partial(
    pl.kernel,
    out_type=[
        jax.ShapeDtypeStruct((NP * W,), jnp.int32),
        jax.ShapeDtypeStruct((NP,), jnp.float32),
    ],
    mesh=_MESH,
    compiler_params=pltpu.CompilerParams(needs_layout_passes=False),
    scratch_types=[
        pltpu.VMEM((RPT * W + 16,), jnp.int32),
        pltpu.VMEM((AE,), jnp.int32),
        pltpu.VMEM((AE,), jnp.int32),
        pltpu.VMEM((AE + 16,), jnp.int32),
        pltpu.VMEM((AE + 16,), jnp.int32),
        pltpu.VMEM((RPT,), jnp.float32),
    ],
)
def _sc_adj_kernel(src_hbm, dst_hbm, pa_hbm, deg_hbm,
                   pa_blk, srcs_v, dsts_v, comp_w, comp_b, degf):
    wid = lax.axis_index("s") * 2 + lax.axis_index("c")
    base = wid * RPT
    hi = base + RPT
    iota16 = lax.iota(jnp.int32, 16)
    zero16 = jnp.zeros((16,), jnp.int32)

    def initpa(i, carry):
        pa_blk[pl.ds(i * 16, 16)] = zero16
        return carry

    lax.fori_loop(0, RPT * W // 16, initpa, 0)

    def chunk_body(ci, carry):
        off = ci * AE
        pltpu.sync_copy(src_hbm.at[pl.ds(off, AE)], srcs_v)
        pltpu.sync_copy(dst_hbm.at[pl.ds(off, AE)], dsts_v)

        for direction in (0, 1):
            a_v, b_v = (srcs_v, dsts_v) if direction == 0 else (dsts_v, srcs_v)

            def filt(i, cnt):
                s = a_v[pl.ds(i * 16, 16)]
                d = b_v[pl.ds(i * 16, 16)]
                m = (s >= base) & (s < hi)
                tot = plsc.all_reduce_population_count(m)[0]

                @pl.when(tot > 0)
                def _():
                    wv = (s - base) * W + lax.shift_right_logical(d, 5)
                    bv = jnp.left_shift(jnp.full((16,), 1, jnp.int32),
                                        d & 31)
                    rel, _t = _prefix16(m, iota16)
                    pos = rel + cnt
                    plsc.store_scatter(comp_w, [pos], wv, mask=m)
                    plsc.store_scatter(comp_b, [pos], bv, mask=m)

                return cnt + tot

            kc = lax.fori_loop(0, AE // 16, filt, jnp.int32(0))
            dump = jnp.full((16,), RPT * W, jnp.int32) + iota16
            zero = jnp.zeros((16,), jnp.int32)

            def rmw_batch(b, carry2):
                eoff = b * 16
                valid = (eoff + iota16) < kc
                wv = jnp.where(valid, comp_w[pl.ds(eoff, 16)], dump)
                bv = jnp.where(valid, comp_b[pl.ds(eoff, 16)], zero)
                sk, sv = plsc.sort_key_val(wv, bv)
                for k in (1, 2, 4, 8):
                    pk = jnp.maximum(iota16 - k, 0)
                    sel = (iota16 >= k) & (sk == _take16(sk, pk))
                    sv = sv | jnp.where(sel, _take16(sv, pk), zero)
                nxt = _take16(sk, jnp.minimum(iota16 + 1, 15))
                last = (sk != nxt) | (iota16 == 15)
                cur = plsc.load_gather(pa_blk, [sk])
                plsc.store_scatter(pa_blk, [sk], cur | sv, mask=last)
                return carry2

            lax.fori_loop(0, lax.shift_right_logical(kc + 15, 4),
                          rmw_batch, 0)
        return carry

    lax.fori_loop(0, EP // AE, chunk_body, 0)

    wmax = jnp.full((16,), WU, jnp.int32)

    def degrow(g, carry):
        r16 = (g * 16 + iota16) * W
        acc = zero16

        def dstep(j, a):
            for u in range(8):
                c = j * 8 + u + iota16
                c = jnp.where(c >= wmax, c - wmax, c)
                wv = plsc.load_gather(pa_blk, [r16 + c])
                a = a + _popcount16(wv)
            return a

        acc = lax.fori_loop(0, WU // 8, dstep, acc)
        degf[pl.ds(g * 16, 16)] = acc.astype(jnp.float32)
        return carry

    lax.fori_loop(0, RPT // 16, degrow, 0)
    pltpu.sync_copy(degf, deg_hbm.at[pl.ds(base, RPT)])
    pltpu.sync_copy(pa_blk.at[pl.ds(0, RPT * W)],
                    pa_hbm.at[pl.ds(base * W, RPT * W)])


@functools.partial(
    pl.kernel,
    out_type=jax.ShapeDtypeStruct((EP,), jnp.float32),
    mesh=_MESH,
    compiler_params=pltpu.CompilerParams(needs_layout_passes=False),
    scratch_types=[
        pltpu.VMEM((EPT + 16,), jnp.int32),
        pltpu.VMEM((EPT + 16,), jnp.int32),
        pltpu.VMEM((EPT,), jnp.float32),
        pltpu.VMEM((NP,), jnp.float32),
        pltpu.VMEM((16,), jnp.int32),
        pltpu.VMEM((16,), jnp.int32),
        pltpu.VMEM((16,), jnp.int32),
        pltpu.VMEM((16,), jnp.int32),
        pltpu.VMEM((16, W), jnp.int32),
        pltpu.VMEM((16, W), jnp.int32),
        pltpu.VMEM((16, W), jnp.int32),
        pltpu.VMEM((16, W), jnp.int32),
        pltpu.SemaphoreType.DMA,
        pltpu.SemaphoreType.DMA,
    ],
)
def _sc_jac_kernel(pa_hbm, deg_hbm, src_hbm, dst_hbm, jac_hbm,
                   sbuf, dbuf, jbuf, degs_v, idx_u0, idx_v0, idx_u1, idx_v1,
                   urows0, vrows0, urows1, vrows1, sem0, sem1):
    wid = lax.axis_index("s") * 2 + lax.axis_index("c")
    base_e = wid * EPT
    iota16 = lax.iota(jnp.int32, 16)
    wmax = jnp.full((16,), WU, jnp.int32)
    zero16 = jnp.zeros((16,), jnp.int32)

    sbuf[pl.ds(EPT, 16)] = zero16
    dbuf[pl.ds(EPT, 16)] = zero16
    pltpu.sync_copy(src_hbm.at[pl.ds(base_e, EPT)], sbuf.at[pl.ds(0, EPT)])
    pltpu.sync_copy(dst_hbm.at[pl.ds(base_e, EPT)], dbuf.at[pl.ds(0, EPT)])
    pltpu.sync_copy(deg_hbm, degs_v)

    bufs = ((idx_u0, idx_v0, urows0, vrows0, sem0),
            (idx_u1, idx_v1, urows1, vrows1, sem1))

    def issue(b, bs):
        iu, iv, ur, vr, sem = bs
        eoff = b * 16
        iu[...] = sbuf[pl.ds(eoff, 16)]
        iv[...] = dbuf[pl.ds(eoff, 16)]
        pltpu.async_copy(pa_hbm.at[iu], ur, sem)
        pltpu.async_copy(pa_hbm.at[iv], vr, sem)

    def process(b, bs):
        iu, iv, ur, vr, sem = bs
        pltpu.make_async_copy(pa_hbm.at[iu], ur, sem).wait()
        pltpu.make_async_copy(pa_hbm.at[iv], vr, sem).wait()

        def wstep(j, a):
            for u in range(8):
                c = j * 8 + u + iota16
                c = jnp.where(c >= wmax, c - wmax, c)
                uw = plsc.load_gather(ur, [iota16, c])
                vw = plsc.load_gather(vr, [iota16, c])
                a = a + _popcount16(uw & vw)
            return a

        acc = lax.fori_loop(0, WU // 8, wstep, jnp.zeros((16,), jnp.int32))
        it = acc.astype(jnp.float32)
        du = plsc.load_gather(degs_v, [iu[...]])
        dv = plsc.load_gather(degs_v, [iv[...]])
        jbuf[pl.ds(b * 16, 16)] = it / (du + dv - it)

    issue(0, bufs[0])

    def pair(g, carry):
        b0 = g * 2
        issue(b0 + 1, bufs[1])
        process(b0, bufs[0])
        issue(b0 + 2, bufs[0])
        process(b0 + 1, bufs[1])
        return carry

    lax.fori_loop(0, EPT // 32, pair, 0)
    pltpu.make_async_copy(pa_hbm.at[idx_u0], urows0, sem0).wait()
    pltpu.make_async_copy(pa_hbm.at[idx_v0], vrows0, sem0).wait()
    pltpu.sync_copy(jbuf, jac_hbm.at[pl.ds(base_e, EPT)])


@functools.partial(
    pl.kernel,
    out_type=jax.ShapeDtypeStruct((NP * D,), jnp.float32),
    mesh=_MESH,
    compiler_params=pltpu.CompilerParams(needs_layout_passes=False),
    scratch_types=[
        pltpu.VMEM((RPT * D,), jnp.float32),
        pltpu.VMEM((NP,), jnp.float32),
        pltpu.VMEM((D,), jnp.float32),
        pltpu.VMEM((16,), jnp.float32),
        pltpu.VMEM((CE,), jnp.int32),
        pltpu.VMEM((CE,), jnp.int32),
        pltpu.VMEM((CE,), jnp.float32),
        pltpu.VMEM((CE,), jnp.int32),
        pltpu.VMEM((CE,), jnp.int32),
        pltpu.VMEM((CE,), jnp.float32),
        pltpu.VMEM((16,), jnp.int32),
        pltpu.VMEM((16,), jnp.int32),
        pltpu.VMEM((16,), jnp.int32),
        pltpu.VMEM((16,), jnp.int32),
        pltpu.VMEM((16, D), jnp.float32),
        pltpu.VMEM((16, D), jnp.float32),
        pltpu.VMEM((16, D), jnp.float32),
        pltpu.VMEM((16, D), jnp.float32),
        pltpu.SemaphoreType.DMA,
        pltpu.SemaphoreType.DMA,
    ],
)
def _sc_msg_kernel(y_hbm, ns_hbm, src_hbm, dst_hbm, jac_hbm, bias_hbm,
                   cvec_hbm, h_hbm,
                   acc_v, ns_v, bias_v, cv_v, srcs_v, dsts_v, jacs_v,
                   comp_src, comp_dst, comp_jac, idx_s0, idx_t0, idx_s1,
                   idx_t1, srows0, trows0, srows1, trows1, sem0, sem1):
    wid = lax.axis_index("s") * 2 + lax.axis_index("c")
    base = wid * RPT
    hi = base + RPT
    iota16 = lax.iota(jnp.int32, 16)

    neg_inf = jnp.full((16,), -jnp.inf, jnp.float32)

    def initacc(i, carry):
        acc_v[pl.ds(i * 16, 16)] = neg_inf
        return carry

    lax.fori_loop(0, RPT * D // 16, initacc, 0)

    zero16 = jnp.zeros((16,), jnp.int32)

    def initcomp(i, carry):
        comp_src[pl.ds(i * 16, 16)] = zero16
        comp_dst[pl.ds(i * 16, 16)] = zero16
        return carry

    lax.fori_loop(0, CE // 16, initcomp, 0)

    pltpu.sync_copy(ns_hbm, ns_v)
    pltpu.sync_copy(bias_hbm, bias_v)
    pltpu.sync_copy(cvec_hbm, cv_v)
    cvec = cv_v[...]
    bvs = [bias_v[pl.ds(k * 16, 16)] for k in range(D // 16)]

    def chunk_body(ci, carry):
        off = ci * CE
        pltpu.sync_copy(src_hbm.at[pl.ds(off, CE)], srcs_v)
        pltpu.sync_copy(dst_hbm.at[pl.ds(off, CE)], dsts_v)
        pltpu.sync_copy(jac_hbm.at[pl.ds(off, CE)], jacs_v)

        def filt(i, cnt):
            d = dsts_v[pl.ds(i * 16, 16)]
            m = (d >= base) & (d < hi)
            tot = plsc.all_reduce_population_count(m)[0]

            @pl.when(tot > 0)
            def _():
                s = srcs_v[pl.ds(i * 16, 16)]
                j = jacs_v[pl.ds(i * 16, 16)]
                rel, _t = _prefix16(m, iota16)
                pos = rel + cnt
                plsc.store_scatter(comp_src, [pos], s, mask=m)
                plsc.store_scatter(comp_dst, [pos], d, mask=m)
                plsc.store_scatter(comp_jac, [pos], j, mask=m)

            return cnt + tot

        kc = lax.fori_loop(0, CE // 16, filt, jnp.int32(0))
        nb = lax.shift_right_logical(kc + 15, 4)

        bufs = ((idx_s0, idx_t0, srows0, trows0, sem0),
                (idx_s1, idx_t1, srows1, trows1, sem1))

        def issue(b, bs):
            i_s, i_t, sr, tr, sem = bs
            eoff = b * 16
            i_s[...] = comp_src[pl.ds(eoff, 16)]
            i_t[...] = comp_dst[pl.ds(eoff, 16)]
            pltpu.async_copy(y_hbm.at[i_s], sr, sem)
            pltpu.async_copy(y_hbm.at[i_t], tr, sem)

        def process(b, bs):
            i_s, i_t, sr, tr, sem = bs
            pltpu.make_async_copy(y_hbm.at[i_s], sr, sem).wait()
            pltpu.make_async_copy(y_hbm.at[i_t], tr, sem).wait()
            eoff = b * 16

            def dot_step(j, dvec):
                for u in range(8):
                    col = (j + u + iota16) & (D - 1)
                    sj = plsc.load_gather(sr, [iota16, col])
                    tj = plsc.load_gather(tr, [iota16, col])
                    dvec = dvec + sj * tj
                return dvec

            dv = lax.fori_loop(0, D // 8, lambda a, v: dot_step(a * 8, v),
                               jnp.zeros((16,), jnp.float32))
            ns_s = plsc.load_gather(ns_v, [i_s[...]])
            ns_t = plsc.load_gather(ns_v, [i_t[...]])
            sim = dv / (ns_s * ns_t)
            jv = comp_jac[pl.ds(eoff, 16)]
            scale = (1.0 - cvec) * jv + cvec * sim
            dstv = comp_dst[pl.ds(eoff, 16)] - base

            def rmw(e, carry3):
                el16 = iota16 * 0 + e
                dloc16 = _take16(dstv, el16) * D
                scv = _take16(scale, el16)
                for k in range(D // 16):
                    ci = k * 16 + iota16
                    addr = dloc16 + ci
                    msg = scv * plsc.load_gather(sr, [el16, ci])
                    cur = plsc.load_gather(acc_v, [addr])
                    plsc.store_scatter(acc_v, [addr], jnp.maximum(cur, msg))
                return carry3

            lax.fori_loop(0, jnp.minimum(kc - eoff, 16), rmw, 0)

        @pl.when(nb > 0)
        def _():
            issue(0, bufs[0])

        def pairbody(g, carry2):
            b0 = g * 2

            @pl.when(b0 + 1 < nb)
            def _():
                issue(b0 + 1, bufs[1])

            process(b0, bufs[0])

            @pl.when(b0 + 2 < nb)
            def _():
                issue(b0 + 2, bufs[0])

            @pl.when(b0 + 1 < nb)
            def _():
                process(b0 + 1, bufs[1])

            return carry2

        lax.fori_loop(0, lax.shift_right_logical(nb + 1, 1), pairbody, 0)
        return carry

    if _STAGE >= 2:
        lax.fori_loop(0, EP // CE, chunk_body, 0)

    def flush(r, carry):
        for k in range(D // 16):
            o = r * D + k * 16
            acc_v[pl.ds(o, 16)] = jnp.maximum(acc_v[pl.ds(o, 16)] + bvs[k], 0.0)
        return carry

    lax.fori_loop(0, RPT, flush, 0)
    pltpu.sync_copy(acc_v, h_hbm.at[pl.ds(base * D, RPT * D)])


def _gcn_layer_sc(y, ns, src_p, dstc_p, jac_p, b, c):
    cvec = jnp.broadcast_to(c.astype(jnp.float32), (16,))
    h = _sc_msg_kernel(y, ns, src_p, dstc_p, jac_p, b, cvec)
    return h.reshape(NP, D)


def _compute_jaccard(src, dst, n):
    A = jnp.zeros((n, n), dtype=bool).at[src, dst].set(True)
    A = A | A.T
    deg = jnp.sum(A, axis=1).astype(jnp.float32)
    e_tot = src.shape[0]
    chunk = 2048
    n_chunks = (e_tot + chunk - 1) // chunk
    pad = n_chunks * chunk - e_tot
    sp = jnp.pad(src, (0, pad)).reshape(n_chunks, chunk)
    dp = jnp.pad(dst, (0, pad)).reshape(n_chunks, chunk)

    def body(sd):
        s, d = sd
        return jnp.sum(A[s] & A[d], axis=1).astype(jnp.float32)

    inter = jax.lax.map(body, (sp, dp)).reshape(-1)[:e_tot]
    union = deg[src] + deg[dst] - inter
    return inter / union


def _gcn_layer_jax(y, ns, src, dst, jac, b, c):
    s = y[src]
    t = y[dst]
    sim = jnp.sum(s * t, axis=-1) / (ns[src] * ns[dst])
    nrm = (1.0 - c) * jac + c * sim
    msgs = nrm[:, None] * s
    out = jax.ops.segment_max(msgs, dst, num_segments=NP)
    return jax.nn.relu(out + b)


_COMPILE_ONLY_SC = False


def kernel(x, edge_index, W1, b1, c1, W2, b2, c2, W_out, b_out):
    if _COMPILE_ONLY_SC:
        loops = jnp.arange(N, dtype=edge_index.dtype)
        src = jnp.concatenate([edge_index[0], loops])
        dst = jnp.concatenate([edge_index[1], loops])
        src_p = jnp.zeros((EP,), jnp.int32).at[:E_REAL].set(src)
        dstc_p = jnp.full((EP,), 1 << 20, jnp.int32).at[:E_REAL].set(dst)
        jac_p = jnp.zeros((EP,), jnp.float32)
        xp = jnp.zeros((NP, D), jnp.float32).at[:N].set(x)
        ns = jnp.ones((NP,), jnp.float32)
        h = _gcn_layer_sc(xp, ns, src_p, dstc_p, jac_p, b1, c1)
        return h[:N, :N_CLS]

    loops = jnp.arange(N, dtype=edge_index.dtype)
    src = jnp.concatenate([edge_index[0], loops])
    dst = jnp.concatenate([edge_index[1], loops])
    src_p = jnp.zeros((EP,), jnp.int32).at[:E_REAL].set(src)
    dst_p = jnp.zeros((EP,), jnp.int32).at[:E_REAL].set(dst)

    xp = jnp.zeros((NP, D), jnp.float32).at[:N].set(x)

    dstc_p = jnp.full((EP,), 1 << 20, jnp.int32).at[:E_REAL].set(dst)

    pa_flat, degs = _sc_adj_kernel(src_p, dst_p)
    jac_p = _sc_jac_kernel(pa_flat.reshape(NP, W), degs, src_p, dst_p)

    y1, ns1 = _linear(xp, W1)
    h1 = _gcn_layer_sc(y1, ns1[:, 0], src_p, dstc_p, jac_p, b1, c1)
    y2, ns2 = _linear(h1, W2)
    h2 = _gcn_layer_sc(y2, ns2[:, 0], src_p, dstc_p, jac_p, b2, c2)
    return _head(h2, W_out, b_out)


# speedup vs baseline: 1.2392x; 1.2392x over previous
import functools

import jax
import jax.numpy as jnp
from jax import lax
from jax.experimental import pallas as pl
from jax.experimental.pallas import tpu as pltpu
from jax.experimental.pallas import tpu_sc as plsc

N = 10000
D = 128
N_CLS = 64
NP = 10240
RPT = NP // 32
W = 384
WU = 320
E_REAL = 170000
EP = 172032
EPT = EP // 32


def _lin_kernel(x_ref, w_ref, y_ref, ns_ref):
    y = jnp.dot(x_ref[...], w_ref[...].T, preferred_element_type=jnp.float32)
    y_ref[...] = y
    ns = jnp.sqrt(jnp.sum(y * y, axis=-1, keepdims=True))
    ns_ref[...] = jnp.maximum(ns, 1e-6)


def _linear(x, Wm):
    return pl.pallas_call(
        _lin_kernel,
        grid=(NP // 512,),
        in_specs=[
            pl.BlockSpec((512, D), lambda i: (i, 0)),
            pl.BlockSpec((D, D), lambda i: (0, 0)),
        ],
        out_specs=[
            pl.BlockSpec((512, D), lambda i: (i, 0)),
            pl.BlockSpec((512, 1), lambda i: (i, 0)),
        ],
        out_shape=[
            jax.ShapeDtypeStruct((NP, D), jnp.float32),
            jax.ShapeDtypeStruct((NP, 1), jnp.float32),
        ],
    )(x, Wm)


def _head_kernel(h_ref, w_ref, b_ref, o_ref):
    logits = jnp.dot(h_ref[...], w_ref[...].T,
                     preferred_element_type=jnp.float32) + b_ref[...]
    m = jnp.max(logits, axis=-1, keepdims=True)
    z = logits - m
    lse = jnp.log(jnp.sum(jnp.exp(z), axis=-1, keepdims=True))
    o_ref[...] = z - lse


def _head(h, W_out, b_out):
    out = pl.pallas_call(
        _head_kernel,
        grid=(NP // 512,),
        in_specs=[
            pl.BlockSpec((512, D), lambda i: (i, 0)),
            pl.BlockSpec((N_CLS, D), lambda i: (0, 0)),
            pl.BlockSpec((1, N_CLS), lambda i: (0, 0)),
        ],
        out_specs=pl.BlockSpec((512, N_CLS), lambda i: (i, 0)),
        out_shape=jax.ShapeDtypeStruct((NP, N_CLS), jnp.float32),
    )(h, W_out, b_out.reshape(1, N_CLS))
    return out[:N]


CE = 8192
_STAGE = 5
_MESH = plsc.VectorSubcoreMesh(core_axis_name="c", subcore_axis_name="s")

_DNUMS = lax.GatherDimensionNumbers(
    offset_dims=(), collapsed_slice_dims=(0,), start_index_map=(0,))


def _take16(v, idx):
    return lax.gather(v, idx[:, None], _DNUMS, (1,),
                      mode=lax.GatherScatterMode.PROMISE_IN_BOUNDS)


def _prefix16(m, iota16):
    zero = jnp.zeros((16,), jnp.int32)
    v = jnp.where(m, jnp.full((16,), 1, jnp.int32), zero)
    for k in (1, 2, 4, 8):
        g = _take16(v, jnp.maximum(iota16 - k, 0))
        v = v + jnp.where(iota16 >= k, g, zero)
    return v - 1, v[15]


def _popcount16(v):
    c55 = jnp.full((16,), 0x55555555, jnp.int32)
    c33 = jnp.full((16,), 0x33333333, jnp.int32)
    c0f = jnp.full((16,), 0x0F0F0F0F, jnp.int32)
    c01 = jnp.full((16,), 0x01010101, jnp.int32)
    v = v - (lax.shift_right_logical(v, 1) & c55)
    v = (v & c33) + (lax.shift_right_logical(v, 2) & c33)
    v = (v + lax.shift_right_logical(v, 4)) & c0f
    return lax.shift_right_logical(v * c01, 24)


AE = 512


@functools.partial(
    pl.kernel,
    out_type=[
        jax.ShapeDtypeStruct((NP * W,), jnp.int32),
        jax.ShapeDtypeStruct((NP,), jnp.float32),
    ],
    mesh=_MESH,
    compiler_params=pltpu.CompilerParams(needs_layout_passes=False),
    scratch_types=[
        pltpu.VMEM((RPT * W + 16,), jnp.int32),
        pltpu.VMEM((AE,), jnp.int32),
        pltpu.VMEM((AE,), jnp.int32),
        pltpu.VMEM((AE + 16,), jnp.int32),
        pltpu.VMEM((AE + 16,), jnp.int32),
        pltpu.VMEM((RPT,), jnp.float32),
    ],
)
def _sc_adj_kernel(src_hbm, dst_hbm, pa_hbm, deg_hbm,
                   pa_blk, srcs_v, dsts_v, comp_w, comp_b, degf):
    wid = lax.axis_index("s") * 2 + lax.axis_index("c")
    base = wid * RPT
    hi = base + RPT
    iota16 = lax.iota(jnp.int32, 16)
    zero16 = jnp.zeros((16,), jnp.int32)

    def initpa(i, carry):
        pa_blk[pl.ds(i * 16, 16)] = zero16
        return carry

    lax.fori_loop(0, RPT * W // 16, initpa, 0)

    def chunk_body(ci, carry):
        off = ci * AE
        pltpu.sync_copy(src_hbm.at[pl.ds(off, AE)], srcs_v)
        pltpu.sync_copy(dst_hbm.at[pl.ds(off, AE)], dsts_v)

        for direction in (0, 1):
            a_v, b_v = (srcs_v, dsts_v) if direction == 0 else (dsts_v, srcs_v)

            def filt(i, cnt):
                s = a_v[pl.ds(i * 16, 16)]
                d = b_v[pl.ds(i * 16, 16)]
                m = (s >= base) & (s < hi)
                wv = (s - base) * W + lax.shift_right_logical(d, 5)
                bv = jnp.left_shift(jnp.full((16,), 1, jnp.int32), d & 31)
                rel, tot = _prefix16(m, iota16)
                pos = rel + cnt
                plsc.store_scatter(comp_w, [pos], wv, mask=m)
                plsc.store_scatter(comp_b, [pos], bv, mask=m)
                return cnt + tot

            kc = lax.fori_loop(0, AE // 16, filt, jnp.int32(0))
            dump = jnp.full((16,), RPT * W, jnp.int32) + iota16
            zero = jnp.zeros((16,), jnp.int32)

            def rmw_batch(b, carry2):
                eoff = b * 16
                valid = (eoff + iota16) < kc
                wv = jnp.where(valid, comp_w[pl.ds(eoff, 16)], dump)
                bv = jnp.where(valid, comp_b[pl.ds(eoff, 16)], zero)
                sk, sv = plsc.sort_key_val(wv, bv)
                for k in (1, 2, 4, 8):
                    pk = jnp.maximum(iota16 - k, 0)
                    sel = (iota16 >= k) & (sk == _take16(sk, pk))
                    sv = sv | jnp.where(sel, _take16(sv, pk), zero)
                nxt = _take16(sk, jnp.minimum(iota16 + 1, 15))
                last = (sk != nxt) | (iota16 == 15)
                cur = plsc.load_gather(pa_blk, [sk])
                plsc.store_scatter(pa_blk, [sk], cur | sv, mask=last)
                return carry2

            lax.fori_loop(0, lax.shift_right_logical(kc + 15, 4),
                          rmw_batch, 0)
        return carry

    lax.fori_loop(0, EP // AE, chunk_body, 0)

    wmax = jnp.full((16,), WU, jnp.int32)

    def degrow(g, carry):
        r16 = (g * 16 + iota16) * W
        acc = zero16

        def dstep(j, a):
            for u in range(8):
                c = j * 8 + u + iota16
                c = jnp.where(c >= wmax, c - wmax, c)
                wv = plsc.load_gather(pa_blk, [r16 + c])
                a = a + _popcount16(wv)
            return a

        acc = lax.fori_loop(0, WU // 8, dstep, acc)
        degf[pl.ds(g * 16, 16)] = acc.astype(jnp.float32)
        return carry

    lax.fori_loop(0, RPT // 16, degrow, 0)
    pltpu.sync_copy(degf, deg_hbm.at[pl.ds(base, RPT)])
    pltpu.sync_copy(pa_blk.at[pl.ds(0, RPT * W)],
                    pa_hbm.at[pl.ds(base * W, RPT * W)])


@functools.partial(
    pl.kernel,
    out_type=jax.ShapeDtypeStruct((EP,), jnp.float32),
    mesh=_MESH,
    compiler_params=pltpu.CompilerParams(needs_layout_passes=False),
    scratch_types=[
        pltpu.VMEM((EPT + 16,), jnp.int32),
        pltpu.VMEM((EPT + 16,), jnp.int32),
        pltpu.VMEM((EPT,), jnp.float32),
        pltpu.VMEM((NP,), jnp.float32),
        pltpu.VMEM((16,), jnp.int32),
        pltpu.VMEM((16,), jnp.int32),
        pltpu.VMEM((16,), jnp.int32),
        pltpu.VMEM((16,), jnp.int32),
        pltpu.VMEM((16, W), jnp.int32),
        pltpu.VMEM((16, W), jnp.int32),
        pltpu.VMEM((16, W), jnp.int32),
        pltpu.VMEM((16, W), jnp.int32),
        pltpu.SemaphoreType.DMA,
        pltpu.SemaphoreType.DMA,
    ],
)
def _sc_jac_kernel(pa_hbm, deg_hbm, src_hbm, dst_hbm, jac_hbm,
                   sbuf, dbuf, jbuf, degs_v, idx_u0, idx_v0, idx_u1, idx_v1,
                   urows0, vrows0, urows1, vrows1, sem0, sem1):
    wid = lax.axis_index("s") * 2 + lax.axis_index("c")
    base_e = wid * EPT
    iota16 = lax.iota(jnp.int32, 16)
    wmax = jnp.full((16,), WU, jnp.int32)
    zero16 = jnp.zeros((16,), jnp.int32)

    sbuf[pl.ds(EPT, 16)] = zero16
    dbuf[pl.ds(EPT, 16)] = zero16
    pltpu.sync_copy(src_hbm.at[pl.ds(base_e, EPT)], sbuf.at[pl.ds(0, EPT)])
    pltpu.sync_copy(dst_hbm.at[pl.ds(base_e, EPT)], dbuf.at[pl.ds(0, EPT)])
    pltpu.sync_copy(deg_hbm, degs_v)

    bufs = ((idx_u0, idx_v0, urows0, vrows0, sem0),
            (idx_u1, idx_v1, urows1, vrows1, sem1))

    def issue(b, bs):
        iu, iv, ur, vr, sem = bs
        eoff = b * 16
        iu[...] = sbuf[pl.ds(eoff, 16)]
        iv[...] = dbuf[pl.ds(eoff, 16)]
        pltpu.async_copy(pa_hbm.at[iu], ur, sem)
        pltpu.async_copy(pa_hbm.at[iv], vr, sem)

    def process(b, bs):
        iu, iv, ur, vr, sem = bs
        pltpu.make_async_copy(pa_hbm.at[iu], ur, sem).wait()
        pltpu.make_async_copy(pa_hbm.at[iv], vr, sem).wait()

        def wstep(j, a):
            for u in range(8):
                c = j * 8 + u + iota16
                c = jnp.where(c >= wmax, c - wmax, c)
                uw = plsc.load_gather(ur, [iota16, c])
                vw = plsc.load_gather(vr, [iota16, c])
                a = a + _popcount16(uw & vw)
            return a

        acc = lax.fori_loop(0, WU // 8, wstep, jnp.zeros((16,), jnp.int32))
        it = acc.astype(jnp.float32)
        du = plsc.load_gather(degs_v, [iu[...]])
        dv = plsc.load_gather(degs_v, [iv[...]])
        jbuf[pl.ds(b * 16, 16)] = it / (du + dv - it)

    issue(0, bufs[0])

    def pair(g, carry):
        b0 = g * 2
        issue(b0 + 1, bufs[1])
        process(b0, bufs[0])
        issue(b0 + 2, bufs[0])
        process(b0 + 1, bufs[1])
        return carry

    lax.fori_loop(0, EPT // 32, pair, 0)
    pltpu.make_async_copy(pa_hbm.at[idx_u0], urows0, sem0).wait()
    pltpu.make_async_copy(pa_hbm.at[idx_v0], vrows0, sem0).wait()
    pltpu.sync_copy(jbuf, jac_hbm.at[pl.ds(base_e, EPT)])


@functools.partial(
    pl.kernel,
    out_type=jax.ShapeDtypeStruct((NP * D,), jnp.float32),
    mesh=_MESH,
    compiler_params=pltpu.CompilerParams(needs_layout_passes=False),
    scratch_types=[
        pltpu.VMEM((RPT * D,), jnp.float32),
        pltpu.VMEM((NP,), jnp.float32),
        pltpu.VMEM((D,), jnp.float32),
        pltpu.VMEM((16,), jnp.float32),
        pltpu.VMEM((CE,), jnp.int32),
        pltpu.VMEM((CE,), jnp.int32),
        pltpu.VMEM((CE,), jnp.float32),
        pltpu.VMEM((CE,), jnp.int32),
        pltpu.VMEM((CE,), jnp.int32),
        pltpu.VMEM((CE,), jnp.float32),
        pltpu.VMEM((16,), jnp.int32),
        pltpu.VMEM((16,), jnp.int32),
        pltpu.VMEM((16,), jnp.int32),
        pltpu.VMEM((16,), jnp.int32),
        pltpu.VMEM((16, D), jnp.float32),
        pltpu.VMEM((16, D), jnp.float32),
        pltpu.VMEM((16, D), jnp.float32),
        pltpu.VMEM((16, D), jnp.float32),
        pltpu.SemaphoreType.DMA,
        pltpu.SemaphoreType.DMA,
    ],
)
def _sc_msg_kernel(y_hbm, ns_hbm, src_hbm, dst_hbm, jac_hbm, bias_hbm,
                   cvec_hbm, h_hbm,
                   acc_v, ns_v, bias_v, cv_v, srcs_v, dsts_v, jacs_v,
                   comp_src, comp_dst, comp_jac, idx_s0, idx_t0, idx_s1,
                   idx_t1, srows0, trows0, srows1, trows1, sem0, sem1):
    wid = lax.axis_index("s") * 2 + lax.axis_index("c")
    base = wid * RPT
    hi = base + RPT
    iota16 = lax.iota(jnp.int32, 16)

    neg_inf = jnp.full((16,), -jnp.inf, jnp.float32)

    def initacc(i, carry):
        acc_v[pl.ds(i * 16, 16)] = neg_inf
        return carry

    lax.fori_loop(0, RPT * D // 16, initacc, 0)

    zero16 = jnp.zeros((16,), jnp.int32)

    def initcomp(i, carry):
        comp_src[pl.ds(i * 16, 16)] = zero16
        comp_dst[pl.ds(i * 16, 16)] = zero16
        return carry

    lax.fori_loop(0, CE // 16, initcomp, 0)

    pltpu.sync_copy(ns_hbm, ns_v)
    pltpu.sync_copy(bias_hbm, bias_v)
    pltpu.sync_copy(cvec_hbm, cv_v)
    cvec = cv_v[...]
    bvs = [bias_v[pl.ds(k * 16, 16)] for k in range(D // 16)]

    def chunk_body(ci, carry):
        off = ci * CE
        pltpu.sync_copy(src_hbm.at[pl.ds(off, CE)], srcs_v)
        pltpu.sync_copy(dst_hbm.at[pl.ds(off, CE)], dsts_v)
        pltpu.sync_copy(jac_hbm.at[pl.ds(off, CE)], jacs_v)

        def filt(i, cnt):
            s = srcs_v[pl.ds(i * 16, 16)]
            d = dsts_v[pl.ds(i * 16, 16)]
            j = jacs_v[pl.ds(i * 16, 16)]
            m = (d >= base) & (d < hi)
            rel, tot = _prefix16(m, iota16)
            pos = rel + cnt
            plsc.store_scatter(comp_src, [pos], s, mask=m)
            plsc.store_scatter(comp_dst, [pos], d, mask=m)
            plsc.store_scatter(comp_jac, [pos], j, mask=m)
            return cnt + tot

        kc = lax.fori_loop(0, CE // 16, filt, jnp.int32(0))
        nb = lax.shift_right_logical(kc + 15, 4)

        bufs = ((idx_s0, idx_t0, srows0, trows0, sem0),
                (idx_s1, idx_t1, srows1, trows1, sem1))

        def issue(b, bs):
            i_s, i_t, sr, tr, sem = bs
            eoff = b * 16
            i_s[...] = comp_src[pl.ds(eoff, 16)]
            i_t[...] = comp_dst[pl.ds(eoff, 16)]
            pltpu.async_copy(y_hbm.at[i_s], sr, sem)
            pltpu.async_copy(y_hbm.at[i_t], tr, sem)

        def process(b, bs):
            i_s, i_t, sr, tr, sem = bs
            pltpu.make_async_copy(y_hbm.at[i_s], sr, sem).wait()
            pltpu.make_async_copy(y_hbm.at[i_t], tr, sem).wait()
            eoff = b * 16

            def dot_step(j, dvec):
                for u in range(8):
                    col = (j + u + iota16) & (D - 1)
                    sj = plsc.load_gather(sr, [iota16, col])
                    tj = plsc.load_gather(tr, [iota16, col])
                    dvec = dvec + sj * tj
                return dvec

            dv = lax.fori_loop(0, D // 8, lambda a, v: dot_step(a * 8, v),
                               jnp.zeros((16,), jnp.float32))
            ns_s = plsc.load_gather(ns_v, [i_s[...]])
            ns_t = plsc.load_gather(ns_v, [i_t[...]])
            sim = dv / (ns_s * ns_t)
            jv = comp_jac[pl.ds(eoff, 16)]
            scale = (1.0 - cvec) * jv + cvec * sim
            dstv = comp_dst[pl.ds(eoff, 16)] - base

            def rmw(e, carry3):
                el16 = iota16 * 0 + e
                dloc16 = _take16(dstv, el16) * D
                scv = _take16(scale, el16)
                for k in range(D // 16):
                    ci = k * 16 + iota16
                    addr = dloc16 + ci
                    msg = scv * plsc.load_gather(sr, [el16, ci])
                    cur = plsc.load_gather(acc_v, [addr])
                    plsc.store_scatter(acc_v, [addr], jnp.maximum(cur, msg))
                return carry3

            lax.fori_loop(0, jnp.minimum(kc - eoff, 16), rmw, 0)

        @pl.when(nb > 0)
        def _():
            issue(0, bufs[0])

        def pairbody(g, carry2):
            b0 = g * 2

            @pl.when(b0 + 1 < nb)
            def _():
                issue(b0 + 1, bufs[1])

            process(b0, bufs[0])

            @pl.when(b0 + 2 < nb)
            def _():
                issue(b0 + 2, bufs[0])

            @pl.when(b0 + 1 < nb)
            def _():
                process(b0 + 1, bufs[1])

            return carry2

        lax.fori_loop(0, lax.shift_right_logical(nb + 1, 1), pairbody, 0)
        return carry

    if _STAGE >= 2:
        lax.fori_loop(0, EP // CE, chunk_body, 0)

    def flush(r, carry):
        for k in range(D // 16):
            o = r * D + k * 16
            acc_v[pl.ds(o, 16)] = jnp.maximum(acc_v[pl.ds(o, 16)] + bvs[k], 0.0)
        return carry

    lax.fori_loop(0, RPT, flush, 0)
    pltpu.sync_copy(acc_v, h_hbm.at[pl.ds(base * D, RPT * D)])


def _gcn_layer_sc(y, ns, src_p, dstc_p, jac_p, b, c):
    cvec = jnp.broadcast_to(c.astype(jnp.float32), (16,))
    h = _sc_msg_kernel(y, ns, src_p, dstc_p, jac_p, b, cvec)
    return h.reshape(NP, D)


def _compute_jaccard(src, dst, n):
    A = jnp.zeros((n, n), dtype=bool).at[src, dst].set(True)
    A = A | A.T
    deg = jnp.sum(A, axis=1).astype(jnp.float32)
    e_tot = src.shape[0]
    chunk = 2048
    n_chunks = (e_tot + chunk - 1) // chunk
    pad = n_chunks * chunk - e_tot
    sp = jnp.pad(src, (0, pad)).reshape(n_chunks, chunk)
    dp = jnp.pad(dst, (0, pad)).reshape(n_chunks, chunk)

    def body(sd):
        s, d = sd
        return jnp.sum(A[s] & A[d], axis=1).astype(jnp.float32)

    inter = jax.lax.map(body, (sp, dp)).reshape(-1)[:e_tot]
    union = deg[src] + deg[dst] - inter
    return inter / union


def _gcn_layer_jax(y, ns, src, dst, jac, b, c):
    s = y[src]
    t = y[dst]
    sim = jnp.sum(s * t, axis=-1) / (ns[src] * ns[dst])
    nrm = (1.0 - c) * jac + c * sim
    msgs = nrm[:, None] * s
    out = jax.ops.segment_max(msgs, dst, num_segments=NP)
    return jax.nn.relu(out + b)


_COMPILE_ONLY_SC = False


def kernel(x, edge_index, W1, b1, c1, W2, b2, c2, W_out, b_out):
    if _COMPILE_ONLY_SC:
        loops = jnp.arange(N, dtype=edge_index.dtype)
        src = jnp.concatenate([edge_index[0], loops])
        dst = jnp.concatenate([edge_index[1], loops])
        src_p = jnp.zeros((EP,), jnp.int32).at[:E_REAL].set(src)
        dstc_p = jnp.full((EP,), 1 << 20, jnp.int32).at[:E_REAL].set(dst)
        jac_p = jnp.zeros((EP,), jnp.float32)
        xp = jnp.zeros((NP, D), jnp.float32).at[:N].set(x)
        ns = jnp.ones((NP,), jnp.float32)
        h = _gcn_layer_sc(xp, ns, src_p, dstc_p, jac_p, b1, c1)
        return h[:N, :N_CLS]

    loops = jnp.arange(N, dtype=edge_index.dtype)
    src = jnp.concatenate([edge_index[0], loops])
    dst = jnp.concatenate([edge_index[1], loops])
    src_p = jnp.zeros((EP,), jnp.int32).at[:E_REAL].set(src)
    dst_p = jnp.zeros((EP,), jnp.int32).at[:E_REAL].set(dst)

    xp = jnp.zeros((NP, D), jnp.float32).at[:N].set(x)

    dstc_p = jnp.full((EP,), 1 << 20, jnp.int32).at[:E_REAL].set(dst)

    pa_flat, degs = _sc_adj_kernel(src_p, dst_p)
    jac_p = _sc_jac_kernel(pa_flat.reshape(NP, W), degs, src_p, dst_p)

    y1, ns1 = _linear(xp, W1)
    h1 = _gcn_layer_sc(y1, ns1[:, 0], src_p, dstc_p, jac_p, b1, c1)
    y2, ns2 = _linear(h1, W2)
    h2 = _gcn_layer_sc(y2, ns2[:, 0], src_p, dstc_p, jac_p, b2, c2)
    return _head(h2, W_out, b_out)


# speedup vs baseline: 1.4258x; 1.1506x over previous
import functools

import jax
import jax.numpy as jnp
from jax import lax
from jax.experimental import pallas as pl
from jax.experimental.pallas import tpu as pltpu
from jax.experimental.pallas import tpu_sc as plsc

N = 10000
D = 128
N_CLS = 64
NP = 10240
RPT = NP // 32
W = 384
WU = 320
E_REAL = 170000
EP = 172032
EPT = EP // 32


def _lin_kernel(x_ref, w_ref, y_ref, ns_ref):
    y = jnp.dot(x_ref[...], w_ref[...].T, preferred_element_type=jnp.float32)
    y_ref[...] = y
    ns = jnp.sqrt(jnp.sum(y * y, axis=-1, keepdims=True))
    ns_ref[...] = jnp.maximum(ns, 1e-6)


def _linear(x, Wm):
    return pl.pallas_call(
        _lin_kernel,
        grid=(NP // 512,),
        in_specs=[
            pl.BlockSpec((512, D), lambda i: (i, 0)),
            pl.BlockSpec((D, D), lambda i: (0, 0)),
        ],
        out_specs=[
            pl.BlockSpec((512, D), lambda i: (i, 0)),
            pl.BlockSpec((512, 1), lambda i: (i, 0)),
        ],
        out_shape=[
            jax.ShapeDtypeStruct((NP, D), jnp.float32),
            jax.ShapeDtypeStruct((NP, 1), jnp.float32),
        ],
    )(x, Wm)


def _head_kernel(h_ref, w_ref, b_ref, o_ref):
    logits = jnp.dot(h_ref[...], w_ref[...].T,
                     preferred_element_type=jnp.float32) + b_ref[...]
    m = jnp.max(logits, axis=-1, keepdims=True)
    z = logits - m
    lse = jnp.log(jnp.sum(jnp.exp(z), axis=-1, keepdims=True))
    o_ref[...] = z - lse


def _head(h, W_out, b_out):
    out = pl.pallas_call(
        _head_kernel,
        grid=(NP // 512,),
        in_specs=[
            pl.BlockSpec((512, D), lambda i: (i, 0)),
            pl.BlockSpec((N_CLS, D), lambda i: (0, 0)),
            pl.BlockSpec((1, N_CLS), lambda i: (0, 0)),
        ],
        out_specs=pl.BlockSpec((512, N_CLS), lambda i: (i, 0)),
        out_shape=jax.ShapeDtypeStruct((NP, N_CLS), jnp.float32),
    )(h, W_out, b_out.reshape(1, N_CLS))
    return out[:N]


CE = 8192
_STAGE = 5
_MESH = plsc.VectorSubcoreMesh(core_axis_name="c", subcore_axis_name="s")

_DNUMS = lax.GatherDimensionNumbers(
    offset_dims=(), collapsed_slice_dims=(0,), start_index_map=(0,))


def _take16(v, idx):
    return lax.gather(v, idx[:, None], _DNUMS, (1,),
                      mode=lax.GatherScatterMode.PROMISE_IN_BOUNDS)


def _prefix16(m, iota16):
    zero = jnp.zeros((16,), jnp.int32)
    v = jnp.where(m, jnp.full((16,), 1, jnp.int32), zero)
    for k in (1, 2, 4, 8):
        g = _take16(v, jnp.maximum(iota16 - k, 0))
        v = v + jnp.where(iota16 >= k, g, zero)
    return v - 1, v[15]


def _popcount16(v):
    c55 = jnp.full((16,), 0x55555555, jnp.int32)
    c33 = jnp.full((16,), 0x33333333, jnp.int32)
    c0f = jnp.full((16,), 0x0F0F0F0F, jnp.int32)
    c01 = jnp.full((16,), 0x01010101, jnp.int32)
    v = v - (lax.shift_right_logical(v, 1) & c55)
    v = (v & c33) + (lax.shift_right_logical(v, 2) & c33)
    v = (v + lax.shift_right_logical(v, 4)) & c0f
    return lax.shift_right_logical(v * c01, 24)


AE = 512


@functools.partial(
    pl.kernel,
    out_type=[
        jax.ShapeDtypeStruct((NP * W,), jnp.int32),
        jax.ShapeDtypeStruct((NP,), jnp.float32),
    ],
    mesh=_MESH,
    compiler_params=pltpu.CompilerParams(needs_layout_passes=False),
    scratch_types=[
        pltpu.VMEM((RPT * W + 16,), jnp.int32),
        pltpu.VMEM((AE,), jnp.int32),
        pltpu.VMEM((AE,), jnp.int32),
        pltpu.VMEM((AE,), jnp.int32),
        pltpu.VMEM((AE,), jnp.int32),
        pltpu.VMEM((AE + 16,), jnp.int32),
        pltpu.VMEM((AE + 16,), jnp.int32),
        pltpu.VMEM((RPT,), jnp.float32),
        pltpu.SemaphoreType.DMA,
        pltpu.SemaphoreType.DMA,
    ],
)
def _sc_adj_kernel(src_hbm, dst_hbm, pa_hbm, deg_hbm,
                   pa_blk, srcs0, dsts0, srcs1, dsts1, comp_w, comp_b, degf,
                   semc0, semc1):
    wid = lax.axis_index("s") * 2 + lax.axis_index("c")
    base = wid * RPT
    hi = base + RPT
    iota16 = lax.iota(jnp.int32, 16)
    zero16 = jnp.zeros((16,), jnp.int32)

    def initpa(i, carry):
        pa_blk[pl.ds(i * 16, 16)] = zero16
        return carry

    lax.fori_loop(0, RPT * W // 16, initpa, 0)

    NCH = EP // AE
    csets = ((srcs0, dsts0, semc0), (srcs1, dsts1, semc1))

    def issue_chunk(ci, cs):
        sv, dv, sem = cs
        off = ci * AE
        pltpu.async_copy(src_hbm.at[pl.ds(off, AE)], sv, sem)
        pltpu.async_copy(dst_hbm.at[pl.ds(off, AE)], dv, sem)

    def process_chunk(ci, cs):
        srcs_v, dsts_v, sem = cs
        off = ci * AE
        pltpu.make_async_copy(src_hbm.at[pl.ds(off, AE)], srcs_v, sem).wait()
        pltpu.make_async_copy(dst_hbm.at[pl.ds(off, AE)], dsts_v, sem).wait()

        for direction in (0, 1):
            a_v, b_v = (srcs_v, dsts_v) if direction == 0 else (dsts_v, srcs_v)

            def filt(i, cnt):
                s = a_v[pl.ds(i * 16, 16)]
                d = b_v[pl.ds(i * 16, 16)]
                m = (s >= base) & (s < hi)
                wv = (s - base) * W + lax.shift_right_logical(d, 5)
                bv = jnp.left_shift(jnp.full((16,), 1, jnp.int32), d & 31)
                rel, tot = _prefix16(m, iota16)
                pos = rel + cnt
                plsc.store_scatter(comp_w, [pos], wv, mask=m)
                plsc.store_scatter(comp_b, [pos], bv, mask=m)
                return cnt + tot

            kc = lax.fori_loop(0, AE // 16, filt, jnp.int32(0))
            dump = jnp.full((16,), RPT * W, jnp.int32) + iota16
            zero = jnp.zeros((16,), jnp.int32)

            def rmw_batch(b, carry2):
                eoff = b * 16
                valid = (eoff + iota16) < kc
                wv = jnp.where(valid, comp_w[pl.ds(eoff, 16)], dump)
                bv = jnp.where(valid, comp_b[pl.ds(eoff, 16)], zero)
                sk, sv = plsc.sort_key_val(wv, bv)
                for k in (1, 2, 4, 8):
                    pk = jnp.maximum(iota16 - k, 0)
                    sel = (iota16 >= k) & (sk == _take16(sk, pk))
                    sv = sv | jnp.where(sel, _take16(sv, pk), zero)
                nxt = _take16(sk, jnp.minimum(iota16 + 1, 15))
                last = (sk != nxt) | (iota16 == 15)
                cur = plsc.load_gather(pa_blk, [sk])
                plsc.store_scatter(pa_blk, [sk], cur | sv, mask=last)
                return carry2

            lax.fori_loop(0, lax.shift_right_logical(kc + 15, 4),
                          rmw_batch, 0)

    issue_chunk(0, csets[0])

    def pairc(g, carry):
        c0 = g * 2
        issue_chunk(c0 + 1, csets[1])
        process_chunk(c0, csets[0])
        issue_chunk(jnp.minimum(c0 + 2, NCH - 1), csets[0])
        process_chunk(c0 + 1, csets[1])
        return carry

    lax.fori_loop(0, NCH // 2, pairc, 0)
    pltpu.make_async_copy(src_hbm.at[pl.ds(0, AE)], srcs0, semc0).wait()
    pltpu.make_async_copy(dst_hbm.at[pl.ds(0, AE)], dsts0, semc0).wait()

    wmax = jnp.full((16,), WU, jnp.int32)

    def degrow(g, carry):
        r16 = (g * 16 + iota16) * W
        acc = zero16

        def dstep(j, a):
            for u in range(8):
                c = j * 8 + u + iota16
                c = jnp.where(c >= wmax, c - wmax, c)
                wv = plsc.load_gather(pa_blk, [r16 + c])
                a = a + _popcount16(wv)
            return a

        acc = lax.fori_loop(0, WU // 8, dstep, acc)
        degf[pl.ds(g * 16, 16)] = acc.astype(jnp.float32)
        return carry

    lax.fori_loop(0, RPT // 16, degrow, 0)
    pltpu.sync_copy(degf, deg_hbm.at[pl.ds(base, RPT)])
    pltpu.sync_copy(pa_blk.at[pl.ds(0, RPT * W)],
                    pa_hbm.at[pl.ds(base * W, RPT * W)])


@functools.partial(
    pl.kernel,
    out_type=jax.ShapeDtypeStruct((EP,), jnp.float32),
    mesh=_MESH,
    compiler_params=pltpu.CompilerParams(needs_layout_passes=False),
    scratch_types=[
        pltpu.VMEM((EPT + 16,), jnp.int32),
        pltpu.VMEM((EPT + 16,), jnp.int32),
        pltpu.VMEM((EPT,), jnp.float32),
        pltpu.VMEM((NP,), jnp.float32),
        pltpu.VMEM((16,), jnp.int32),
        pltpu.VMEM((16,), jnp.int32),
        pltpu.VMEM((16,), jnp.int32),
        pltpu.VMEM((16,), jnp.int32),
        pltpu.VMEM((16, W), jnp.int32),
        pltpu.VMEM((16, W), jnp.int32),
        pltpu.VMEM((16, W), jnp.int32),
        pltpu.VMEM((16, W), jnp.int32),
        pltpu.SemaphoreType.DMA,
        pltpu.SemaphoreType.DMA,
    ],
)
def _sc_jac_kernel(pa_hbm, deg_hbm, src_hbm, dst_hbm, jac_hbm,
                   sbuf, dbuf, jbuf, degs_v, idx_u0, idx_v0, idx_u1, idx_v1,
                   urows0, vrows0, urows1, vrows1, sem0, sem1):
    wid = lax.axis_index("s") * 2 + lax.axis_index("c")
    base_e = wid * EPT
    iota16 = lax.iota(jnp.int32, 16)
    wmax = jnp.full((16,), WU, jnp.int32)
    zero16 = jnp.zeros((16,), jnp.int32)

    sbuf[pl.ds(EPT, 16)] = zero16
    dbuf[pl.ds(EPT, 16)] = zero16
    pltpu.sync_copy(src_hbm.at[pl.ds(base_e, EPT)], sbuf.at[pl.ds(0, EPT)])
    pltpu.sync_copy(dst_hbm.at[pl.ds(base_e, EPT)], dbuf.at[pl.ds(0, EPT)])
    pltpu.sync_copy(deg_hbm, degs_v)

    bufs = ((idx_u0, idx_v0, urows0, vrows0, sem0),
            (idx_u1, idx_v1, urows1, vrows1, sem1))

    def issue(b, bs):
        iu, iv, ur, vr, sem = bs
        eoff = b * 16
        iu[...] = sbuf[pl.ds(eoff, 16)]
        iv[...] = dbuf[pl.ds(eoff, 16)]
        pltpu.async_copy(pa_hbm.at[iu], ur, sem)
        pltpu.async_copy(pa_hbm.at[iv], vr, sem)

    def process(b, bs):
        iu, iv, ur, vr, sem = bs
        pltpu.make_async_copy(pa_hbm.at[iu], ur, sem).wait()
        pltpu.make_async_copy(pa_hbm.at[iv], vr, sem).wait()

        def wstep(j, a):
            for u in range(8):
                c = j * 8 + u + iota16
                c = jnp.where(c >= wmax, c - wmax, c)
                uw = plsc.load_gather(ur, [iota16, c])
                vw = plsc.load_gather(vr, [iota16, c])
                a = a + _popcount16(uw & vw)
            return a

        acc = lax.fori_loop(0, WU // 8, wstep, jnp.zeros((16,), jnp.int32))
        it = acc.astype(jnp.float32)
        du = plsc.load_gather(degs_v, [iu[...]])
        dv = plsc.load_gather(degs_v, [iv[...]])
        jbuf[pl.ds(b * 16, 16)] = it / (du + dv - it)

    issue(0, bufs[0])

    def pair(g, carry):
        b0 = g * 2
        issue(b0 + 1, bufs[1])
        process(b0, bufs[0])
        issue(b0 + 2, bufs[0])
        process(b0 + 1, bufs[1])
        return carry

    lax.fori_loop(0, EPT // 32, pair, 0)
    pltpu.make_async_copy(pa_hbm.at[idx_u0], urows0, sem0).wait()
    pltpu.make_async_copy(pa_hbm.at[idx_v0], vrows0, sem0).wait()
    pltpu.sync_copy(jbuf, jac_hbm.at[pl.ds(base_e, EPT)])


@functools.partial(
    pl.kernel,
    out_type=jax.ShapeDtypeStruct((NP * D,), jnp.float32),
    mesh=_MESH,
    compiler_params=pltpu.CompilerParams(needs_layout_passes=False),
    scratch_types=[
        pltpu.VMEM((RPT * D,), jnp.float32),
        pltpu.VMEM((NP,), jnp.float32),
        pltpu.VMEM((D,), jnp.float32),
        pltpu.VMEM((16,), jnp.float32),
        pltpu.VMEM((CE,), jnp.int32),
        pltpu.VMEM((CE,), jnp.int32),
        pltpu.VMEM((CE,), jnp.float32),
        pltpu.VMEM((CE,), jnp.int32),
        pltpu.VMEM((CE,), jnp.int32),
        pltpu.VMEM((CE,), jnp.float32),
        pltpu.VMEM((16,), jnp.int32),
        pltpu.VMEM((16,), jnp.int32),
        pltpu.VMEM((16,), jnp.int32),
        pltpu.VMEM((16,), jnp.int32),
        pltpu.VMEM((16, D), jnp.float32),
        pltpu.VMEM((16, D), jnp.float32),
        pltpu.VMEM((16, D), jnp.float32),
        pltpu.VMEM((16, D), jnp.float32),
        pltpu.SemaphoreType.DMA,
        pltpu.SemaphoreType.DMA,
    ],
)
def _sc_msg_kernel(y_hbm, ns_hbm, src_hbm, dst_hbm, jac_hbm, bias_hbm,
                   cvec_hbm, h_hbm,
                   acc_v, ns_v, bias_v, cv_v, srcs_v, dsts_v, jacs_v,
                   comp_src, comp_dst, comp_jac, idx_s0, idx_t0, idx_s1,
                   idx_t1, srows0, trows0, srows1, trows1, sem0, sem1):
    wid = lax.axis_index("s") * 2 + lax.axis_index("c")
    base = wid * RPT
    hi = base + RPT
    iota16 = lax.iota(jnp.int32, 16)

    neg_inf = jnp.full((16,), -jnp.inf, jnp.float32)

    def initacc(i, carry):
        acc_v[pl.ds(i * 16, 16)] = neg_inf
        return carry

    lax.fori_loop(0, RPT * D // 16, initacc, 0)

    zero16 = jnp.zeros((16,), jnp.int32)

    def initcomp(i, carry):
        comp_src[pl.ds(i * 16, 16)] = zero16
        comp_dst[pl.ds(i * 16, 16)] = zero16
        return carry

    lax.fori_loop(0, CE // 16, initcomp, 0)

    pltpu.sync_copy(ns_hbm, ns_v)
    pltpu.sync_copy(bias_hbm, bias_v)
    pltpu.sync_copy(cvec_hbm, cv_v)
    cvec = cv_v[...]
    bvs = [bias_v[pl.ds(k * 16, 16)] for k in range(D // 16)]

    def chunk_body(ci, carry):
        off = ci * CE
        pltpu.sync_copy(src_hbm.at[pl.ds(off, CE)], srcs_v)
        pltpu.sync_copy(dst_hbm.at[pl.ds(off, CE)], dsts_v)
        pltpu.sync_copy(jac_hbm.at[pl.ds(off, CE)], jacs_v)

        def filt(i, cnt):
            s = srcs_v[pl.ds(i * 16, 16)]
            d = dsts_v[pl.ds(i * 16, 16)]
            j = jacs_v[pl.ds(i * 16, 16)]
            m = (d >= base) & (d < hi)
            rel, tot = _prefix16(m, iota16)
            pos = rel + cnt
            plsc.store_scatter(comp_src, [pos], s, mask=m)
            plsc.store_scatter(comp_dst, [pos], d, mask=m)
            plsc.store_scatter(comp_jac, [pos], j, mask=m)
            return cnt + tot

        kc = lax.fori_loop(0, CE // 16, filt, jnp.int32(0))
        nb = lax.shift_right_logical(kc + 15, 4)

        bufs = ((idx_s0, idx_t0, srows0, trows0, sem0),
                (idx_s1, idx_t1, srows1, trows1, sem1))

        def issue(b, bs):
            i_s, i_t, sr, tr, sem = bs
            eoff = b * 16
            i_s[...] = comp_src[pl.ds(eoff, 16)]
            i_t[...] = comp_dst[pl.ds(eoff, 16)]
            pltpu.async_copy(y_hbm.at[i_s], sr, sem)
            pltpu.async_copy(y_hbm.at[i_t], tr, sem)

        def process(b, bs):
            i_s, i_t, sr, tr, sem = bs
            pltpu.make_async_copy(y_hbm.at[i_s], sr, sem).wait()
            pltpu.make_async_copy(y_hbm.at[i_t], tr, sem).wait()
            eoff = b * 16

            def dot_step(j, dvec):
                for u in range(8):
                    col = (j + u + iota16) & (D - 1)
                    sj = plsc.load_gather(sr, [iota16, col])
                    tj = plsc.load_gather(tr, [iota16, col])
                    dvec = dvec + sj * tj
                return dvec

            dv = lax.fori_loop(0, D // 8, lambda a, v: dot_step(a * 8, v),
                               jnp.zeros((16,), jnp.float32))
            ns_s = plsc.load_gather(ns_v, [i_s[...]])
            ns_t = plsc.load_gather(ns_v, [i_t[...]])
            sim = dv / (ns_s * ns_t)
            jv = comp_jac[pl.ds(eoff, 16)]
            scale = (1.0 - cvec) * jv + cvec * sim
            dstv = comp_dst[pl.ds(eoff, 16)] - base

            def rmw(e, carry3):
                el16 = iota16 * 0 + e
                dloc16 = _take16(dstv, el16) * D
                scv = _take16(scale, el16)
                for k in range(D // 16):
                    ci = k * 16 + iota16
                    addr = dloc16 + ci
                    msg = scv * plsc.load_gather(sr, [el16, ci])
                    cur = plsc.load_gather(acc_v, [addr])
                    plsc.store_scatter(acc_v, [addr], jnp.maximum(cur, msg))
                return carry3

            lax.fori_loop(0, jnp.minimum(kc - eoff, 16), rmw, 0)

        @pl.when(nb > 0)
        def _():
            issue(0, bufs[0])

        def pairbody(g, carry2):
            b0 = g * 2

            @pl.when(b0 + 1 < nb)
            def _():
                issue(b0 + 1, bufs[1])

            process(b0, bufs[0])

            @pl.when(b0 + 2 < nb)
            def _():
                issue(b0 + 2, bufs[0])

            @pl.when(b0 + 1 < nb)
            def _():
                process(b0 + 1, bufs[1])

            return carry2

        lax.fori_loop(0, lax.shift_right_logical(nb + 1, 1), pairbody, 0)
        return carry

    if _STAGE >= 2:
        lax.fori_loop(0, EP // CE, chunk_body, 0)

    def flush(r, carry):
        for k in range(D // 16):
            o = r * D + k * 16
            acc_v[pl.ds(o, 16)] = jnp.maximum(acc_v[pl.ds(o, 16)] + bvs[k], 0.0)
        return carry

    lax.fori_loop(0, RPT, flush, 0)
    pltpu.sync_copy(acc_v, h_hbm.at[pl.ds(base * D, RPT * D)])


def _gcn_layer_sc(y, ns, src_p, dstc_p, jac_p, b, c):
    cvec = jnp.broadcast_to(c.astype(jnp.float32), (16,))
    h = _sc_msg_kernel(y, ns, src_p, dstc_p, jac_p, b, cvec)
    return h.reshape(NP, D)


def _compute_jaccard(src, dst, n):
    A = jnp.zeros((n, n), dtype=bool).at[src, dst].set(True)
    A = A | A.T
    deg = jnp.sum(A, axis=1).astype(jnp.float32)
    e_tot = src.shape[0]
    chunk = 2048
    n_chunks = (e_tot + chunk - 1) // chunk
    pad = n_chunks * chunk - e_tot
    sp = jnp.pad(src, (0, pad)).reshape(n_chunks, chunk)
    dp = jnp.pad(dst, (0, pad)).reshape(n_chunks, chunk)

    def body(sd):
        s, d = sd
        return jnp.sum(A[s] & A[d], axis=1).astype(jnp.float32)

    inter = jax.lax.map(body, (sp, dp)).reshape(-1)[:e_tot]
    union = deg[src] + deg[dst] - inter
    return inter / union


def _gcn_layer_jax(y, ns, src, dst, jac, b, c):
    s = y[src]
    t = y[dst]
    sim = jnp.sum(s * t, axis=-1) / (ns[src] * ns[dst])
    nrm = (1.0 - c) * jac + c * sim
    msgs = nrm[:, None] * s
    out = jax.ops.segment_max(msgs, dst, num_segments=NP)
    return jax.nn.relu(out + b)


_COMPILE_ONLY_SC = False


def kernel(x, edge_index, W1, b1, c1, W2, b2, c2, W_out, b_out):
    if _COMPILE_ONLY_SC:
        loops = jnp.arange(N, dtype=edge_index.dtype)
        src = jnp.concatenate([edge_index[0], loops])
        dst = jnp.concatenate([edge_index[1], loops])
        src_p = jnp.zeros((EP,), jnp.int32).at[:E_REAL].set(src)
        dstc_p = jnp.full((EP,), 1 << 20, jnp.int32).at[:E_REAL].set(dst)
        jac_p = jnp.zeros((EP,), jnp.float32)
        xp = jnp.zeros((NP, D), jnp.float32).at[:N].set(x)
        ns = jnp.ones((NP,), jnp.float32)
        h = _gcn_layer_sc(xp, ns, src_p, dstc_p, jac_p, b1, c1)
        return h[:N, :N_CLS]

    loops = jnp.arange(N, dtype=edge_index.dtype)
    src = jnp.concatenate([edge_index[0], loops])
    dst = jnp.concatenate([edge_index[1], loops])
    src_p = jnp.zeros((EP,), jnp.int32).at[:E_REAL].set(src)
    dst_p = jnp.zeros((EP,), jnp.int32).at[:E_REAL].set(dst)

    xp = jnp.zeros((NP, D), jnp.float32).at[:N].set(x)

    dstc_p = jnp.full((EP,), 1 << 20, jnp.int32).at[:E_REAL].set(dst)

    pa_flat, degs = _sc_adj_kernel(src_p, dst_p)
    jac_p = _sc_jac_kernel(pa_flat.reshape(NP, W), degs, src_p, dst_p)

    y1, ns1 = _linear(xp, W1)
    h1 = _gcn_layer_sc(y1, ns1[:, 0], src_p, dstc_p, jac_p, b1, c1)
    y2, ns2 = _linear(h1, W2)
    h2 = _gcn_layer_sc(y2, ns2[:, 0], src_p, dstc_p, jac_p, b2, c2)
    return _head(h2, W_out, b_out)


# speedup vs baseline: 1.4732x; 1.0333x over previous
import functools

import jax
import jax.numpy as jnp
from jax import lax
from jax.experimental import pallas as pl
from jax.experimental.pallas import tpu as pltpu
from jax.experimental.pallas import tpu_sc as plsc

N = 10000
D = 128
N_CLS = 64
NP = 10240
RPT = NP // 32
W = 384
WU = 320
E_REAL = 170000
EP = 172032
EPT = EP // 32


def _lin_kernel(x_ref, w_ref, y_ref, ns_ref):
    y = jnp.dot(x_ref[...], w_ref[...].T, preferred_element_type=jnp.float32)
    y_ref[...] = y
    ns = jnp.sqrt(jnp.sum(y * y, axis=-1, keepdims=True))
    ns_ref[...] = jnp.maximum(ns, 1e-6)


def _linear(x, Wm):
    return pl.pallas_call(
        _lin_kernel,
        grid=(NP // 512,),
        in_specs=[
            pl.BlockSpec((512, D), lambda i: (i, 0)),
            pl.BlockSpec((D, D), lambda i: (0, 0)),
        ],
        out_specs=[
            pl.BlockSpec((512, D), lambda i: (i, 0)),
            pl.BlockSpec((512, 1), lambda i: (i, 0)),
        ],
        out_shape=[
            jax.ShapeDtypeStruct((NP, D), jnp.float32),
            jax.ShapeDtypeStruct((NP, 1), jnp.float32),
        ],
    )(x, Wm)


def _head_kernel(h_ref, w_ref, b_ref, o_ref):
    logits = jnp.dot(h_ref[...], w_ref[...].T,
                     preferred_element_type=jnp.float32) + b_ref[...]
    m = jnp.max(logits, axis=-1, keepdims=True)
    z = logits - m
    lse = jnp.log(jnp.sum(jnp.exp(z), axis=-1, keepdims=True))
    o_ref[...] = z - lse


def _head(h, W_out, b_out):
    out = pl.pallas_call(
        _head_kernel,
        grid=(NP // 512,),
        in_specs=[
            pl.BlockSpec((512, D), lambda i: (i, 0)),
            pl.BlockSpec((N_CLS, D), lambda i: (0, 0)),
            pl.BlockSpec((1, N_CLS), lambda i: (0, 0)),
        ],
        out_specs=pl.BlockSpec((512, N_CLS), lambda i: (i, 0)),
        out_shape=jax.ShapeDtypeStruct((NP, N_CLS), jnp.float32),
    )(h, W_out, b_out.reshape(1, N_CLS))
    return out[:N]


CE = 4096
_STAGE = 5
_MESH = plsc.VectorSubcoreMesh(core_axis_name="c", subcore_axis_name="s")

_DNUMS = lax.GatherDimensionNumbers(
    offset_dims=(), collapsed_slice_dims=(0,), start_index_map=(0,))


def _take16(v, idx):
    return lax.gather(v, idx[:, None], _DNUMS, (1,),
                      mode=lax.GatherScatterMode.PROMISE_IN_BOUNDS)


def _prefix16(m, iota16):
    zero = jnp.zeros((16,), jnp.int32)
    v = jnp.where(m, jnp.full((16,), 1, jnp.int32), zero)
    for k in (1, 2, 4, 8):
        g = _take16(v, jnp.maximum(iota16 - k, 0))
        v = v + jnp.where(iota16 >= k, g, zero)
    return v - 1, v[15]


def _popcount16(v):
    c55 = jnp.full((16,), 0x55555555, jnp.int32)
    c33 = jnp.full((16,), 0x33333333, jnp.int32)
    c0f = jnp.full((16,), 0x0F0F0F0F, jnp.int32)
    c01 = jnp.full((16,), 0x01010101, jnp.int32)
    v = v - (lax.shift_right_logical(v, 1) & c55)
    v = (v & c33) + (lax.shift_right_logical(v, 2) & c33)
    v = (v + lax.shift_right_logical(v, 4)) & c0f
    return lax.shift_right_logical(v * c01, 24)


AE = 512


@functools.partial(
    pl.kernel,
    out_type=[
        jax.ShapeDtypeStruct((NP * W,), jnp.int32),
        jax.ShapeDtypeStruct((NP,), jnp.float32),
    ],
    mesh=_MESH,
    compiler_params=pltpu.CompilerParams(needs_layout_passes=False),
    scratch_types=[
        pltpu.VMEM((RPT * W + 16,), jnp.int32),
        pltpu.VMEM((AE,), jnp.int32),
        pltpu.VMEM((AE,), jnp.int32),
        pltpu.VMEM((AE,), jnp.int32),
        pltpu.VMEM((AE,), jnp.int32),
        pltpu.VMEM((AE + 16,), jnp.int32),
        pltpu.VMEM((AE + 16,), jnp.int32),
        pltpu.VMEM((RPT,), jnp.float32),
        pltpu.SemaphoreType.DMA,
        pltpu.SemaphoreType.DMA,
    ],
)
def _sc_adj_kernel(src_hbm, dst_hbm, pa_hbm, deg_hbm,
                   pa_blk, srcs0, dsts0, srcs1, dsts1, comp_w, comp_b, degf,
                   semc0, semc1):
    wid = lax.axis_index("s") * 2 + lax.axis_index("c")
    base = wid * RPT
    hi = base + RPT
    iota16 = lax.iota(jnp.int32, 16)
    zero16 = jnp.zeros((16,), jnp.int32)

    def initpa(i, carry):
        pa_blk[pl.ds(i * 16, 16)] = zero16
        return carry

    lax.fori_loop(0, RPT * W // 16, initpa, 0)

    NCH = EP // AE
    csets = ((srcs0, dsts0, semc0), (srcs1, dsts1, semc1))

    def issue_chunk(ci, cs):
        sv, dv, sem = cs
        off = ci * AE
        pltpu.async_copy(src_hbm.at[pl.ds(off, AE)], sv, sem)
        pltpu.async_copy(dst_hbm.at[pl.ds(off, AE)], dv, sem)

    def process_chunk(ci, cs):
        srcs_v, dsts_v, sem = cs
        off = ci * AE
        pltpu.make_async_copy(src_hbm.at[pl.ds(off, AE)], srcs_v, sem).wait()
        pltpu.make_async_copy(dst_hbm.at[pl.ds(off, AE)], dsts_v, sem).wait()

        for direction in (0, 1):
            a_v, b_v = (srcs_v, dsts_v) if direction == 0 else (dsts_v, srcs_v)

            def filt(i, cnt):
                s = a_v[pl.ds(i * 16, 16)]
                d = b_v[pl.ds(i * 16, 16)]
                m = (s >= base) & (s < hi)
                wv = (s - base) * W + lax.shift_right_logical(d, 5)
                bv = jnp.left_shift(jnp.full((16,), 1, jnp.int32), d & 31)
                rel, tot = _prefix16(m, iota16)
                pos = rel + cnt
                plsc.store_scatter(comp_w, [pos], wv, mask=m)
                plsc.store_scatter(comp_b, [pos], bv, mask=m)
                return cnt + tot

            kc = lax.fori_loop(0, AE // 16, filt, jnp.int32(0))
            dump = jnp.full((16,), RPT * W, jnp.int32) + iota16
            zero = jnp.zeros((16,), jnp.int32)

            def rmw_batch(b, carry2):
                eoff = b * 16
                valid = (eoff + iota16) < kc
                wv = jnp.where(valid, comp_w[pl.ds(eoff, 16)], dump)
                bv = jnp.where(valid, comp_b[pl.ds(eoff, 16)], zero)
                sk, sv = plsc.sort_key_val(wv, bv)
                for k in (1, 2, 4, 8):
                    pk = jnp.maximum(iota16 - k, 0)
                    sel = (iota16 >= k) & (sk == _take16(sk, pk))
                    sv = sv | jnp.where(sel, _take16(sv, pk), zero)
                nxt = _take16(sk, jnp.minimum(iota16 + 1, 15))
                last = (sk != nxt) | (iota16 == 15)
                cur = plsc.load_gather(pa_blk, [sk])
                plsc.store_scatter(pa_blk, [sk], cur | sv, mask=last)
                return carry2

            lax.fori_loop(0, lax.shift_right_logical(kc + 15, 4),
                          rmw_batch, 0)

    issue_chunk(0, csets[0])

    def pairc(g, carry):
        c0 = g * 2
        issue_chunk(c0 + 1, csets[1])
        process_chunk(c0, csets[0])
        issue_chunk(jnp.minimum(c0 + 2, NCH - 1), csets[0])
        process_chunk(c0 + 1, csets[1])
        return carry

    lax.fori_loop(0, NCH // 2, pairc, 0)
    pltpu.make_async_copy(src_hbm.at[pl.ds(0, AE)], srcs0, semc0).wait()
    pltpu.make_async_copy(dst_hbm.at[pl.ds(0, AE)], dsts0, semc0).wait()

    wmax = jnp.full((16,), WU, jnp.int32)

    def degrow(g, carry):
        r16 = (g * 16 + iota16) * W
        acc = zero16

        def dstep(j, a):
            for u in range(8):
                c = j * 8 + u + iota16
                c = jnp.where(c >= wmax, c - wmax, c)
                wv = plsc.load_gather(pa_blk, [r16 + c])
                a = a + _popcount16(wv)
            return a

        acc = lax.fori_loop(0, WU // 8, dstep, acc)
        degf[pl.ds(g * 16, 16)] = acc.astype(jnp.float32)
        return carry

    lax.fori_loop(0, RPT // 16, degrow, 0)
    pltpu.sync_copy(degf, deg_hbm.at[pl.ds(base, RPT)])
    pltpu.sync_copy(pa_blk.at[pl.ds(0, RPT * W)],
                    pa_hbm.at[pl.ds(base * W, RPT * W)])


@functools.partial(
    pl.kernel,
    out_type=jax.ShapeDtypeStruct((EP,), jnp.float32),
    mesh=_MESH,
    compiler_params=pltpu.CompilerParams(needs_layout_passes=False),
    scratch_types=[
        pltpu.VMEM((EPT + 16,), jnp.int32),
        pltpu.VMEM((EPT + 16,), jnp.int32),
        pltpu.VMEM((EPT,), jnp.float32),
        pltpu.VMEM((NP,), jnp.float32),
        pltpu.VMEM((16,), jnp.int32),
        pltpu.VMEM((16,), jnp.int32),
        pltpu.VMEM((16,), jnp.int32),
        pltpu.VMEM((16,), jnp.int32),
        pltpu.VMEM((16, W), jnp.int32),
        pltpu.VMEM((16, W), jnp.int32),
        pltpu.VMEM((16, W), jnp.int32),
        pltpu.VMEM((16, W), jnp.int32),
        pltpu.SemaphoreType.DMA,
        pltpu.SemaphoreType.DMA,
    ],
)
def _sc_jac_kernel(pa_hbm, deg_hbm, src_hbm, dst_hbm, jac_hbm,
                   sbuf, dbuf, jbuf, degs_v, idx_u0, idx_v0, idx_u1, idx_v1,
                   urows0, vrows0, urows1, vrows1, sem0, sem1):
    wid = lax.axis_index("s") * 2 + lax.axis_index("c")
    base_e = wid * EPT
    iota16 = lax.iota(jnp.int32, 16)
    wmax = jnp.full((16,), WU, jnp.int32)
    zero16 = jnp.zeros((16,), jnp.int32)

    sbuf[pl.ds(EPT, 16)] = zero16
    dbuf[pl.ds(EPT, 16)] = zero16
    pltpu.sync_copy(src_hbm.at[pl.ds(base_e, EPT)], sbuf.at[pl.ds(0, EPT)])
    pltpu.sync_copy(dst_hbm.at[pl.ds(base_e, EPT)], dbuf.at[pl.ds(0, EPT)])
    pltpu.sync_copy(deg_hbm, degs_v)

    bufs = ((idx_u0, idx_v0, urows0, vrows0, sem0),
            (idx_u1, idx_v1, urows1, vrows1, sem1))

    def issue(b, bs):
        iu, iv, ur, vr, sem = bs
        eoff = b * 16
        iu[...] = sbuf[pl.ds(eoff, 16)]
        iv[...] = dbuf[pl.ds(eoff, 16)]
        pltpu.async_copy(pa_hbm.at[iu], ur, sem)
        pltpu.async_copy(pa_hbm.at[iv], vr, sem)

    def process(b, bs):
        iu, iv, ur, vr, sem = bs
        pltpu.make_async_copy(pa_hbm.at[iu], ur, sem).wait()
        pltpu.make_async_copy(pa_hbm.at[iv], vr, sem).wait()

        def wstep(j, a):
            for u in range(8):
                c = j * 8 + u + iota16
                c = jnp.where(c >= wmax, c - wmax, c)
                uw = plsc.load_gather(ur, [iota16, c])
                vw = plsc.load_gather(vr, [iota16, c])
                a = a + _popcount16(uw & vw)
            return a

        acc = lax.fori_loop(0, WU // 8, wstep, jnp.zeros((16,), jnp.int32))
        it = acc.astype(jnp.float32)
        du = plsc.load_gather(degs_v, [iu[...]])
        dv = plsc.load_gather(degs_v, [iv[...]])
        jbuf[pl.ds(b * 16, 16)] = it / (du + dv - it)

    issue(0, bufs[0])

    def pair(g, carry):
        b0 = g * 2
        issue(b0 + 1, bufs[1])
        process(b0, bufs[0])
        issue(b0 + 2, bufs[0])
        process(b0 + 1, bufs[1])
        return carry

    lax.fori_loop(0, EPT // 32, pair, 0)
    pltpu.make_async_copy(pa_hbm.at[idx_u0], urows0, sem0).wait()
    pltpu.make_async_copy(pa_hbm.at[idx_v0], vrows0, sem0).wait()
    pltpu.sync_copy(jbuf, jac_hbm.at[pl.ds(base_e, EPT)])


@functools.partial(
    pl.kernel,
    out_type=jax.ShapeDtypeStruct((NP * D,), jnp.float32),
    mesh=_MESH,
    compiler_params=pltpu.CompilerParams(needs_layout_passes=False),
    scratch_types=[
        pltpu.VMEM((RPT * D,), jnp.float32),
        pltpu.VMEM((NP,), jnp.float32),
        pltpu.VMEM((D,), jnp.float32),
        pltpu.VMEM((16,), jnp.float32),
        pltpu.VMEM((CE,), jnp.int32),
        pltpu.VMEM((CE,), jnp.int32),
        pltpu.VMEM((CE,), jnp.float32),
        pltpu.VMEM((CE,), jnp.int32),
        pltpu.VMEM((CE,), jnp.int32),
        pltpu.VMEM((CE,), jnp.float32),
        pltpu.SemaphoreType.DMA,
        pltpu.SemaphoreType.DMA,
        pltpu.VMEM((CE,), jnp.int32),
        pltpu.VMEM((CE,), jnp.int32),
        pltpu.VMEM((CE,), jnp.float32),
        pltpu.VMEM((16,), jnp.int32),
        pltpu.VMEM((16,), jnp.int32),
        pltpu.VMEM((16,), jnp.int32),
        pltpu.VMEM((16,), jnp.int32),
        pltpu.VMEM((16, D), jnp.float32),
        pltpu.VMEM((16, D), jnp.float32),
        pltpu.VMEM((16, D), jnp.float32),
        pltpu.VMEM((16, D), jnp.float32),
        pltpu.SemaphoreType.DMA,
        pltpu.SemaphoreType.DMA,
    ],
)
def _sc_msg_kernel(y_hbm, ns_hbm, src_hbm, dst_hbm, jac_hbm, bias_hbm,
                   cvec_hbm, h_hbm,
                   acc_v, ns_v, bias_v, cv_v, esrc0, edst0, ejac0,
                   esrc1, edst1, ejac1, seme0, seme1,
                   comp_src, comp_dst, comp_jac, idx_s0, idx_t0, idx_s1,
                   idx_t1, srows0, trows0, srows1, trows1, sem0, sem1):
    wid = lax.axis_index("s") * 2 + lax.axis_index("c")
    base = wid * RPT
    hi = base + RPT
    iota16 = lax.iota(jnp.int32, 16)

    neg_inf = jnp.full((16,), -jnp.inf, jnp.float32)

    def initacc(i, carry):
        acc_v[pl.ds(i * 16, 16)] = neg_inf
        return carry

    lax.fori_loop(0, RPT * D // 16, initacc, 0)

    zero16 = jnp.zeros((16,), jnp.int32)

    def initcomp(i, carry):
        comp_src[pl.ds(i * 16, 16)] = zero16
        comp_dst[pl.ds(i * 16, 16)] = zero16
        return carry

    lax.fori_loop(0, CE // 16, initcomp, 0)

    pltpu.sync_copy(ns_hbm, ns_v)
    pltpu.sync_copy(bias_hbm, bias_v)
    pltpu.sync_copy(cvec_hbm, cv_v)
    cvec = cv_v[...]
    bvs = [bias_v[pl.ds(k * 16, 16)] for k in range(D // 16)]

    NCH = EP // CE
    csets = ((esrc0, edst0, ejac0, seme0), (esrc1, edst1, ejac1, seme1))

    def issue_chunk(ci, cs):
        sv, dv, jv_, sem = cs
        off = ci * CE
        pltpu.async_copy(src_hbm.at[pl.ds(off, CE)], sv, sem)
        pltpu.async_copy(dst_hbm.at[pl.ds(off, CE)], dv, sem)
        pltpu.async_copy(jac_hbm.at[pl.ds(off, CE)], jv_, sem)

    def process_chunk(ci, cs):
        srcs_v, dsts_v, jacs_v, sem = cs
        off = ci * CE
        pltpu.make_async_copy(src_hbm.at[pl.ds(off, CE)], srcs_v, sem).wait()
        pltpu.make_async_copy(dst_hbm.at[pl.ds(off, CE)], dsts_v, sem).wait()
        pltpu.make_async_copy(jac_hbm.at[pl.ds(off, CE)], jacs_v, sem).wait()

        def filt(i, cnt):
            s = srcs_v[pl.ds(i * 16, 16)]
            d = dsts_v[pl.ds(i * 16, 16)]
            j = jacs_v[pl.ds(i * 16, 16)]
            m = (d >= base) & (d < hi)
            rel, tot = _prefix16(m, iota16)
            pos = rel + cnt
            plsc.store_scatter(comp_src, [pos], s, mask=m)
            plsc.store_scatter(comp_dst, [pos], d, mask=m)
            plsc.store_scatter(comp_jac, [pos], j, mask=m)
            return cnt + tot

        kc = lax.fori_loop(0, CE // 16, filt, jnp.int32(0))
        nb = lax.shift_right_logical(kc + 15, 4)

        bufs = ((idx_s0, idx_t0, srows0, trows0, sem0),
                (idx_s1, idx_t1, srows1, trows1, sem1))

        def issue(b, bs):
            i_s, i_t, sr, tr, sem = bs
            eoff = b * 16
            i_s[...] = comp_src[pl.ds(eoff, 16)]
            i_t[...] = comp_dst[pl.ds(eoff, 16)]
            pltpu.async_copy(y_hbm.at[i_s], sr, sem)
            pltpu.async_copy(y_hbm.at[i_t], tr, sem)

        def process(b, bs):
            i_s, i_t, sr, tr, sem = bs
            pltpu.make_async_copy(y_hbm.at[i_s], sr, sem).wait()
            pltpu.make_async_copy(y_hbm.at[i_t], tr, sem).wait()
            eoff = b * 16

            def dot_step(j, dvec):
                for u in range(8):
                    col = (j + u + iota16) & (D - 1)
                    sj = plsc.load_gather(sr, [iota16, col])
                    tj = plsc.load_gather(tr, [iota16, col])
                    dvec = dvec + sj * tj
                return dvec

            dv = lax.fori_loop(0, D // 8, lambda a, v: dot_step(a * 8, v),
                               jnp.zeros((16,), jnp.float32))
            ns_s = plsc.load_gather(ns_v, [i_s[...]])
            ns_t = plsc.load_gather(ns_v, [i_t[...]])
            sim = dv / (ns_s * ns_t)
            jv = comp_jac[pl.ds(eoff, 16)]
            scale = (1.0 - cvec) * jv + cvec * sim
            dstv = comp_dst[pl.ds(eoff, 16)] - base

            def rmw(e, carry3):
                el16 = iota16 * 0 + e
                dloc16 = _take16(dstv, el16) * D
                scv = _take16(scale, el16)
                for k in range(D // 16):
                    ci = k * 16 + iota16
                    addr = dloc16 + ci
                    msg = scv * plsc.load_gather(sr, [el16, ci])
                    cur = plsc.load_gather(acc_v, [addr])
                    plsc.store_scatter(acc_v, [addr], jnp.maximum(cur, msg))
                return carry3

            lax.fori_loop(0, jnp.minimum(kc - eoff, 16), rmw, 0)

        @pl.when(nb > 0)
        def _():
            issue(0, bufs[0])

        def pairbody(g, carry2):
            b0 = g * 2

            @pl.when(b0 + 1 < nb)
            def _():
                issue(b0 + 1, bufs[1])

            process(b0, bufs[0])

            @pl.when(b0 + 2 < nb)
            def _():
                issue(b0 + 2, bufs[0])

            @pl.when(b0 + 1 < nb)
            def _():
                process(b0 + 1, bufs[1])

            return carry2

        lax.fori_loop(0, lax.shift_right_logical(nb + 1, 1), pairbody, 0)

    issue_chunk(0, csets[0])

    def pairc(g, carry):
        c0 = g * 2
        issue_chunk(c0 + 1, csets[1])
        process_chunk(c0, csets[0])
        issue_chunk(jnp.minimum(c0 + 2, NCH - 1), csets[0])
        process_chunk(c0 + 1, csets[1])
        return carry

    lax.fori_loop(0, NCH // 2, pairc, 0)
    pltpu.make_async_copy(src_hbm.at[pl.ds(0, CE)], esrc0, seme0).wait()
    pltpu.make_async_copy(dst_hbm.at[pl.ds(0, CE)], edst0, seme0).wait()
    pltpu.make_async_copy(jac_hbm.at[pl.ds(0, CE)], ejac0, seme0).wait()

    def flush(r, carry):
        for k in range(D // 16):
            o = r * D + k * 16
            acc_v[pl.ds(o, 16)] = jnp.maximum(acc_v[pl.ds(o, 16)] + bvs[k], 0.0)
        return carry

    lax.fori_loop(0, RPT, flush, 0)
    pltpu.sync_copy(acc_v, h_hbm.at[pl.ds(base * D, RPT * D)])


def _gcn_layer_sc(y, ns, src_p, dstc_p, jac_p, b, c):
    cvec = jnp.broadcast_to(c.astype(jnp.float32), (16,))
    h = _sc_msg_kernel(y, ns, src_p, dstc_p, jac_p, b, cvec)
    return h.reshape(NP, D)


def _compute_jaccard(src, dst, n):
    A = jnp.zeros((n, n), dtype=bool).at[src, dst].set(True)
    A = A | A.T
    deg = jnp.sum(A, axis=1).astype(jnp.float32)
    e_tot = src.shape[0]
    chunk = 2048
    n_chunks = (e_tot + chunk - 1) // chunk
    pad = n_chunks * chunk - e_tot
    sp = jnp.pad(src, (0, pad)).reshape(n_chunks, chunk)
    dp = jnp.pad(dst, (0, pad)).reshape(n_chunks, chunk)

    def body(sd):
        s, d = sd
        return jnp.sum(A[s] & A[d], axis=1).astype(jnp.float32)

    inter = jax.lax.map(body, (sp, dp)).reshape(-1)[:e_tot]
    union = deg[src] + deg[dst] - inter
    return inter / union


def _gcn_layer_jax(y, ns, src, dst, jac, b, c):
    s = y[src]
    t = y[dst]
    sim = jnp.sum(s * t, axis=-1) / (ns[src] * ns[dst])
    nrm = (1.0 - c) * jac + c * sim
    msgs = nrm[:, None] * s
    out = jax.ops.segment_max(msgs, dst, num_segments=NP)
    return jax.nn.relu(out + b)


_COMPILE_ONLY_SC = False


def kernel(x, edge_index, W1, b1, c1, W2, b2, c2, W_out, b_out):
    if _COMPILE_ONLY_SC:
        loops = jnp.arange(N, dtype=edge_index.dtype)
        src = jnp.concatenate([edge_index[0], loops])
        dst = jnp.concatenate([edge_index[1], loops])
        src_p = jnp.zeros((EP,), jnp.int32).at[:E_REAL].set(src)
        dstc_p = jnp.full((EP,), 1 << 20, jnp.int32).at[:E_REAL].set(dst)
        jac_p = jnp.zeros((EP,), jnp.float32)
        xp = jnp.zeros((NP, D), jnp.float32).at[:N].set(x)
        ns = jnp.ones((NP,), jnp.float32)
        h = _gcn_layer_sc(xp, ns, src_p, dstc_p, jac_p, b1, c1)
        return h[:N, :N_CLS]

    loops = jnp.arange(N, dtype=edge_index.dtype)
    src = jnp.concatenate([edge_index[0], loops])
    dst = jnp.concatenate([edge_index[1], loops])
    src_p = jnp.zeros((EP,), jnp.int32).at[:E_REAL].set(src)
    dst_p = jnp.zeros((EP,), jnp.int32).at[:E_REAL].set(dst)

    xp = jnp.zeros((NP, D), jnp.float32).at[:N].set(x)

    dstc_p = jnp.full((EP,), 1 << 20, jnp.int32).at[:E_REAL].set(dst)

    pa_flat, degs = _sc_adj_kernel(src_p, dst_p)
    jac_p = _sc_jac_kernel(pa_flat.reshape(NP, W), degs, src_p, dst_p)

    y1, ns1 = _linear(xp, W1)
    h1 = _gcn_layer_sc(y1, ns1[:, 0], src_p, dstc_p, jac_p, b1, c1)
    y2, ns2 = _linear(h1, W2)
    h2 = _gcn_layer_sc(y2, ns2[:, 0], src_p, dstc_p, jac_p, b2, c2)
    return _head(h2, W_out, b_out)


# speedup vs baseline: 1.4741x; 1.0006x over previous
import functools

import jax
import jax.numpy as jnp
from jax import lax
from jax.experimental import pallas as pl
from jax.experimental.pallas import tpu as pltpu
from jax.experimental.pallas import tpu_sc as plsc

N = 10000
D = 128
N_CLS = 64
NP = 10240
RPT = NP // 32
W = 384
WU = 320
E_REAL = 170000
EP = 172032
EPT = EP // 32


def _lin_kernel(x_ref, w_ref, y_ref, ns_ref):
    y = jnp.dot(x_ref[...], w_ref[...].T, preferred_element_type=jnp.float32)
    y_ref[...] = y
    ns = jnp.sqrt(jnp.sum(y * y, axis=-1, keepdims=True))
    ns_ref[...] = jnp.maximum(ns, 1e-6)


def _linear(x, Wm):
    return pl.pallas_call(
        _lin_kernel,
        grid=(NP // 512,),
        in_specs=[
            pl.BlockSpec((512, D), lambda i: (i, 0)),
            pl.BlockSpec((D, D), lambda i: (0, 0)),
        ],
        out_specs=[
            pl.BlockSpec((512, D), lambda i: (i, 0)),
            pl.BlockSpec((512, 1), lambda i: (i, 0)),
        ],
        out_shape=[
            jax.ShapeDtypeStruct((NP, D), jnp.float32),
            jax.ShapeDtypeStruct((NP, 1), jnp.float32),
        ],
    )(x, Wm)


def _head_kernel(h_ref, w_ref, b_ref, o_ref):
    logits = jnp.dot(h_ref[...], w_ref[...].T,
                     preferred_element_type=jnp.float32) + b_ref[...]
    m = jnp.max(logits, axis=-1, keepdims=True)
    z = logits - m
    lse = jnp.log(jnp.sum(jnp.exp(z), axis=-1, keepdims=True))
    o_ref[...] = z - lse


def _head(h, W_out, b_out):
    out = pl.pallas_call(
        _head_kernel,
        grid=(NP // 512,),
        in_specs=[
            pl.BlockSpec((512, D), lambda i: (i, 0)),
            pl.BlockSpec((N_CLS, D), lambda i: (0, 0)),
            pl.BlockSpec((1, N_CLS), lambda i: (0, 0)),
        ],
        out_specs=pl.BlockSpec((512, N_CLS), lambda i: (i, 0)),
        out_shape=jax.ShapeDtypeStruct((NP, N_CLS), jnp.float32),
    )(h, W_out, b_out.reshape(1, N_CLS))
    return out[:N]


CE = 4096
_MESH = plsc.VectorSubcoreMesh(core_axis_name="c", subcore_axis_name="s")

_DNUMS = lax.GatherDimensionNumbers(
    offset_dims=(), collapsed_slice_dims=(0,), start_index_map=(0,))


def _take16(v, idx):
    return lax.gather(v, idx[:, None], _DNUMS, (1,),
                      mode=lax.GatherScatterMode.PROMISE_IN_BOUNDS)


def _prefix16(m, iota16):
    zero = jnp.zeros((16,), jnp.int32)
    v = jnp.where(m, jnp.full((16,), 1, jnp.int32), zero)
    for k in (1, 2, 4, 8):
        g = _take16(v, jnp.maximum(iota16 - k, 0))
        v = v + jnp.where(iota16 >= k, g, zero)
    return v - 1, v[15]


def _popcount16(v):
    c55 = jnp.full((16,), 0x55555555, jnp.int32)
    c33 = jnp.full((16,), 0x33333333, jnp.int32)
    c0f = jnp.full((16,), 0x0F0F0F0F, jnp.int32)
    c01 = jnp.full((16,), 0x01010101, jnp.int32)
    v = v - (lax.shift_right_logical(v, 1) & c55)
    v = (v & c33) + (lax.shift_right_logical(v, 2) & c33)
    v = (v + lax.shift_right_logical(v, 4)) & c0f
    return lax.shift_right_logical(v * c01, 24)


AE = 512


@functools.partial(
    pl.kernel,
    out_type=[
        jax.ShapeDtypeStruct((NP * W,), jnp.int32),
        jax.ShapeDtypeStruct((NP,), jnp.float32),
    ],
    mesh=_MESH,
    compiler_params=pltpu.CompilerParams(needs_layout_passes=False),
    scratch_types=[
        pltpu.VMEM((RPT * W + 16,), jnp.int32),
        pltpu.VMEM((AE,), jnp.int32),
        pltpu.VMEM((AE,), jnp.int32),
        pltpu.VMEM((AE,), jnp.int32),
        pltpu.VMEM((AE,), jnp.int32),
        pltpu.VMEM((AE + 16,), jnp.int32),
        pltpu.VMEM((AE + 16,), jnp.int32),
        pltpu.VMEM((RPT,), jnp.float32),
        pltpu.SemaphoreType.DMA,
        pltpu.SemaphoreType.DMA,
    ],
)
def _sc_adj_kernel(src_hbm, dst_hbm, pa_hbm, deg_hbm,
                   pa_blk, srcs0, dsts0, srcs1, dsts1, comp_w, comp_b, degf,
                   semc0, semc1):
    wid = lax.axis_index("s") * 2 + lax.axis_index("c")
    base = wid * RPT
    hi = base + RPT
    iota16 = lax.iota(jnp.int32, 16)
    zero16 = jnp.zeros((16,), jnp.int32)

    def initpa(i, carry):
        pa_blk[pl.ds(i * 16, 16)] = zero16
        return carry

    lax.fori_loop(0, RPT * W // 16, initpa, 0)

    NCH = EP // AE
    csets = ((srcs0, dsts0, semc0), (srcs1, dsts1, semc1))

    def issue_chunk(ci, cs):
        sv, dv, sem = cs
        off = ci * AE
        pltpu.async_copy(src_hbm.at[pl.ds(off, AE)], sv, sem)
        pltpu.async_copy(dst_hbm.at[pl.ds(off, AE)], dv, sem)

    def process_chunk(ci, cs):
        srcs_v, dsts_v, sem = cs
        off = ci * AE
        pltpu.make_async_copy(src_hbm.at[pl.ds(off, AE)], srcs_v, sem).wait()
        pltpu.make_async_copy(dst_hbm.at[pl.ds(off, AE)], dsts_v, sem).wait()

        for direction in (0, 1):
            a_v, b_v = (srcs_v, dsts_v) if direction == 0 else (dsts_v, srcs_v)

            def filt(i, cnt):
                s = a_v[pl.ds(i * 16, 16)]
                d = b_v[pl.ds(i * 16, 16)]
                m = (s >= base) & (s < hi)
                wv = (s - base) * W + lax.shift_right_logical(d, 5)
                bv = jnp.left_shift(jnp.full((16,), 1, jnp.int32), d & 31)
                rel, tot = _prefix16(m, iota16)
                pos = rel + cnt
                plsc.store_scatter(comp_w, [pos], wv, mask=m)
                plsc.store_scatter(comp_b, [pos], bv, mask=m)
                return cnt + tot

            kc = lax.fori_loop(0, AE // 16, filt, jnp.int32(0))
            dump = jnp.full((16,), RPT * W, jnp.int32) + iota16
            zero = jnp.zeros((16,), jnp.int32)

            def rmw_batch(b, carry2):
                eoff = b * 16
                valid = (eoff + iota16) < kc
                wv = jnp.where(valid, comp_w[pl.ds(eoff, 16)], dump)
                bv = jnp.where(valid, comp_b[pl.ds(eoff, 16)], zero)
                sk, sv = plsc.sort_key_val(wv, bv)
                for k in (1, 2, 4, 8):
                    pk = jnp.maximum(iota16 - k, 0)
                    sel = (iota16 >= k) & (sk == _take16(sk, pk))
                    sv = sv | jnp.where(sel, _take16(sv, pk), zero)
                nxt = _take16(sk, jnp.minimum(iota16 + 1, 15))
                last = (sk != nxt) | (iota16 == 15)
                cur = plsc.load_gather(pa_blk, [sk])
                plsc.store_scatter(pa_blk, [sk], cur | sv, mask=last)
                return carry2

            lax.fori_loop(0, lax.shift_right_logical(kc + 15, 4),
                          rmw_batch, 0)

    issue_chunk(0, csets[0])

    def pairc(g, carry):
        c0 = g * 2
        issue_chunk(c0 + 1, csets[1])
        process_chunk(c0, csets[0])
        issue_chunk(jnp.minimum(c0 + 2, NCH - 1), csets[0])
        process_chunk(c0 + 1, csets[1])
        return carry

    lax.fori_loop(0, NCH // 2, pairc, 0)
    pltpu.make_async_copy(src_hbm.at[pl.ds(0, AE)], srcs0, semc0).wait()
    pltpu.make_async_copy(dst_hbm.at[pl.ds(0, AE)], dsts0, semc0).wait()

    wmax = jnp.full((16,), WU, jnp.int32)

    def degrow(g, carry):
        r16 = (g * 16 + iota16) * W
        acc = zero16

        def dstep(j, a):
            for u in range(8):
                c = j * 8 + u + iota16
                c = jnp.where(c >= wmax, c - wmax, c)
                wv = plsc.load_gather(pa_blk, [r16 + c])
                a = a + _popcount16(wv)
            return a

        acc = lax.fori_loop(0, WU // 8, dstep, acc)
        degf[pl.ds(g * 16, 16)] = acc.astype(jnp.float32)
        return carry

    lax.fori_loop(0, RPT // 16, degrow, 0)
    pltpu.sync_copy(degf, deg_hbm.at[pl.ds(base, RPT)])
    pltpu.sync_copy(pa_blk.at[pl.ds(0, RPT * W)],
                    pa_hbm.at[pl.ds(base * W, RPT * W)])


@functools.partial(
    pl.kernel,
    out_type=jax.ShapeDtypeStruct((EP,), jnp.float32),
    mesh=_MESH,
    compiler_params=pltpu.CompilerParams(needs_layout_passes=False),
    scratch_types=[
        pltpu.VMEM((EPT + 16,), jnp.int32),
        pltpu.VMEM((EPT + 16,), jnp.int32),
        pltpu.VMEM((EPT,), jnp.float32),
        pltpu.VMEM((NP,), jnp.float32),
        pltpu.VMEM((16,), jnp.int32),
        pltpu.VMEM((16,), jnp.int32),
        pltpu.VMEM((16,), jnp.int32),
        pltpu.VMEM((16,), jnp.int32),
        pltpu.VMEM((16, W), jnp.int32),
        pltpu.VMEM((16, W), jnp.int32),
        pltpu.VMEM((16, W), jnp.int32),
        pltpu.VMEM((16, W), jnp.int32),
        pltpu.SemaphoreType.DMA,
        pltpu.SemaphoreType.DMA,
    ],
)
def _sc_jac_kernel(pa_hbm, deg_hbm, src_hbm, dst_hbm, jac_hbm,
                   sbuf, dbuf, jbuf, degs_v, idx_u0, idx_v0, idx_u1, idx_v1,
                   urows0, vrows0, urows1, vrows1, sem0, sem1):
    wid = lax.axis_index("s") * 2 + lax.axis_index("c")
    base_e = wid * EPT
    iota16 = lax.iota(jnp.int32, 16)
    wmax = jnp.full((16,), WU, jnp.int32)
    zero16 = jnp.zeros((16,), jnp.int32)

    sbuf[pl.ds(EPT, 16)] = zero16
    dbuf[pl.ds(EPT, 16)] = zero16
    pltpu.sync_copy(src_hbm.at[pl.ds(base_e, EPT)], sbuf.at[pl.ds(0, EPT)])
    pltpu.sync_copy(dst_hbm.at[pl.ds(base_e, EPT)], dbuf.at[pl.ds(0, EPT)])
    pltpu.sync_copy(deg_hbm, degs_v)

    bufs = ((idx_u0, idx_v0, urows0, vrows0, sem0),
            (idx_u1, idx_v1, urows1, vrows1, sem1))

    def issue(b, bs):
        iu, iv, ur, vr, sem = bs
        eoff = b * 16
        iu[...] = sbuf[pl.ds(eoff, 16)]
        iv[...] = dbuf[pl.ds(eoff, 16)]
        pltpu.async_copy(pa_hbm.at[iu], ur, sem)
        pltpu.async_copy(pa_hbm.at[iv], vr, sem)

    def process(b, bs):
        iu, iv, ur, vr, sem = bs
        pltpu.make_async_copy(pa_hbm.at[iu], ur, sem).wait()
        pltpu.make_async_copy(pa_hbm.at[iv], vr, sem).wait()

        def wstep(j, a):
            for u in range(8):
                c = j * 8 + u + iota16
                c = jnp.where(c >= wmax, c - wmax, c)
                uw = plsc.load_gather(ur, [iota16, c])
                vw = plsc.load_gather(vr, [iota16, c])
                a = a + _popcount16(uw & vw)
            return a

        acc = lax.fori_loop(0, WU // 8, wstep, jnp.zeros((16,), jnp.int32))
        it = acc.astype(jnp.float32)
        du = plsc.load_gather(degs_v, [iu[...]])
        dv = plsc.load_gather(degs_v, [iv[...]])
        jbuf[pl.ds(b * 16, 16)] = it / (du + dv - it)

    issue(0, bufs[0])

    def pair(g, carry):
        b0 = g * 2
        issue(b0 + 1, bufs[1])
        process(b0, bufs[0])
        issue(b0 + 2, bufs[0])
        process(b0 + 1, bufs[1])
        return carry

    lax.fori_loop(0, EPT // 32, pair, 0)
    pltpu.make_async_copy(pa_hbm.at[idx_u0], urows0, sem0).wait()
    pltpu.make_async_copy(pa_hbm.at[idx_v0], vrows0, sem0).wait()
    pltpu.sync_copy(jbuf, jac_hbm.at[pl.ds(base_e, EPT)])


@functools.partial(
    pl.kernel,
    out_type=jax.ShapeDtypeStruct((NP * D,), jnp.float32),
    mesh=_MESH,
    compiler_params=pltpu.CompilerParams(needs_layout_passes=False),
    scratch_types=[
        pltpu.VMEM((RPT * D,), jnp.float32),
        pltpu.VMEM((NP,), jnp.float32),
        pltpu.VMEM((D,), jnp.float32),
        pltpu.VMEM((16,), jnp.float32),
        pltpu.VMEM((CE,), jnp.int32),
        pltpu.VMEM((CE,), jnp.int32),
        pltpu.VMEM((CE,), jnp.float32),
        pltpu.VMEM((CE,), jnp.int32),
        pltpu.VMEM((CE,), jnp.int32),
        pltpu.VMEM((CE,), jnp.float32),
        pltpu.SemaphoreType.DMA,
        pltpu.SemaphoreType.DMA,
        pltpu.VMEM((CE,), jnp.int32),
        pltpu.VMEM((CE,), jnp.int32),
        pltpu.VMEM((CE,), jnp.float32),
        pltpu.VMEM((16,), jnp.int32),
        pltpu.VMEM((16,), jnp.int32),
        pltpu.VMEM((16,), jnp.int32),
        pltpu.VMEM((16,), jnp.int32),
        pltpu.VMEM((16, D), jnp.float32),
        pltpu.VMEM((16, D), jnp.float32),
        pltpu.VMEM((16, D), jnp.float32),
        pltpu.VMEM((16, D), jnp.float32),
        pltpu.SemaphoreType.DMA,
        pltpu.SemaphoreType.DMA,
    ],
)
def _sc_msg_kernel(y_hbm, ns_hbm, src_hbm, dst_hbm, jac_hbm, bias_hbm,
                   cvec_hbm, h_hbm,
                   acc_v, ns_v, bias_v, cv_v, esrc0, edst0, ejac0,
                   esrc1, edst1, ejac1, seme0, seme1,
                   comp_src, comp_dst, comp_jac, idx_s0, idx_t0, idx_s1,
                   idx_t1, srows0, trows0, srows1, trows1, sem0, sem1):
    wid = lax.axis_index("s") * 2 + lax.axis_index("c")
    base = wid * RPT
    hi = base + RPT
    iota16 = lax.iota(jnp.int32, 16)

    neg_inf = jnp.full((16,), -jnp.inf, jnp.float32)

    def initacc(i, carry):
        acc_v[pl.ds(i * 16, 16)] = neg_inf
        return carry

    lax.fori_loop(0, RPT * D // 16, initacc, 0)

    zero16 = jnp.zeros((16,), jnp.int32)

    def initcomp(i, carry):
        comp_src[pl.ds(i * 16, 16)] = zero16
        comp_dst[pl.ds(i * 16, 16)] = zero16
        return carry

    lax.fori_loop(0, CE // 16, initcomp, 0)

    pltpu.sync_copy(ns_hbm, ns_v)
    pltpu.sync_copy(bias_hbm, bias_v)
    pltpu.sync_copy(cvec_hbm, cv_v)
    cvec = cv_v[...]
    bvs = [bias_v[pl.ds(k * 16, 16)] for k in range(D // 16)]

    NCH = EP // CE
    csets = ((esrc0, edst0, ejac0, seme0), (esrc1, edst1, ejac1, seme1))

    def issue_chunk(ci, cs):
        sv, dv, jv_, sem = cs
        off = ci * CE
        pltpu.async_copy(src_hbm.at[pl.ds(off, CE)], sv, sem)
        pltpu.async_copy(dst_hbm.at[pl.ds(off, CE)], dv, sem)
        pltpu.async_copy(jac_hbm.at[pl.ds(off, CE)], jv_, sem)

    def process_chunk(ci, cs):
        srcs_v, dsts_v, jacs_v, sem = cs
        off = ci * CE
        pltpu.make_async_copy(src_hbm.at[pl.ds(off, CE)], srcs_v, sem).wait()
        pltpu.make_async_copy(dst_hbm.at[pl.ds(off, CE)], dsts_v, sem).wait()
        pltpu.make_async_copy(jac_hbm.at[pl.ds(off, CE)], jacs_v, sem).wait()

        def filt(i, cnt):
            s = srcs_v[pl.ds(i * 16, 16)]
            d = dsts_v[pl.ds(i * 16, 16)]
            j = jacs_v[pl.ds(i * 16, 16)]
            m = (d >= base) & (d < hi)
            rel, tot = _prefix16(m, iota16)
            pos = rel + cnt
            plsc.store_scatter(comp_src, [pos], s, mask=m)
            plsc.store_scatter(comp_dst, [pos], d, mask=m)
            plsc.store_scatter(comp_jac, [pos], j, mask=m)
            return cnt + tot

        kc = lax.fori_loop(0, CE // 16, filt, jnp.int32(0))
        nb = lax.shift_right_logical(kc + 15, 4)

        bufs = ((idx_s0, idx_t0, srows0, trows0, sem0),
                (idx_s1, idx_t1, srows1, trows1, sem1))

        def issue(b, bs):
            i_s, i_t, sr, tr, sem = bs
            eoff = b * 16
            i_s[...] = comp_src[pl.ds(eoff, 16)]
            i_t[...] = comp_dst[pl.ds(eoff, 16)]
            pltpu.async_copy(y_hbm.at[i_s], sr, sem)
            pltpu.async_copy(y_hbm.at[i_t], tr, sem)

        def process(b, bs):
            i_s, i_t, sr, tr, sem = bs
            pltpu.make_async_copy(y_hbm.at[i_s], sr, sem).wait()
            pltpu.make_async_copy(y_hbm.at[i_t], tr, sem).wait()
            eoff = b * 16

            def dot_step(j, dvec):
                for u in range(8):
                    col = (j + u + iota16) & (D - 1)
                    sj = plsc.load_gather(sr, [iota16, col])
                    tj = plsc.load_gather(tr, [iota16, col])
                    dvec = dvec + sj * tj
                return dvec

            dv = lax.fori_loop(0, D // 8, lambda a, v: dot_step(a * 8, v),
                               jnp.zeros((16,), jnp.float32))
            ns_s = plsc.load_gather(ns_v, [i_s[...]])
            ns_t = plsc.load_gather(ns_v, [i_t[...]])
            sim = dv / (ns_s * ns_t)
            jv = comp_jac[pl.ds(eoff, 16)]
            scale = (1.0 - cvec) * jv + cvec * sim
            dstv = comp_dst[pl.ds(eoff, 16)] - base

            def rmw(e, carry3):
                el16 = iota16 * 0 + e
                dloc16 = _take16(dstv, el16) * D
                scv = _take16(scale, el16)
                for k in range(D // 16):
                    ci = k * 16 + iota16
                    addr = dloc16 + ci
                    msg = scv * plsc.load_gather(sr, [el16, ci])
                    cur = plsc.load_gather(acc_v, [addr])
                    plsc.store_scatter(acc_v, [addr], jnp.maximum(cur, msg))
                return carry3

            lax.fori_loop(0, jnp.minimum(kc - eoff, 16), rmw, 0)

        @pl.when(nb > 0)
        def _():
            issue(0, bufs[0])

        def pairbody(g, carry2):
            b0 = g * 2

            @pl.when(b0 + 1 < nb)
            def _():
                issue(b0 + 1, bufs[1])

            process(b0, bufs[0])

            @pl.when(b0 + 2 < nb)
            def _():
                issue(b0 + 2, bufs[0])

            @pl.when(b0 + 1 < nb)
            def _():
                process(b0 + 1, bufs[1])

            return carry2

        lax.fori_loop(0, lax.shift_right_logical(nb + 1, 1), pairbody, 0)

    issue_chunk(0, csets[0])

    def pairc(g, carry):
        c0 = g * 2
        issue_chunk(c0 + 1, csets[1])
        process_chunk(c0, csets[0])
        issue_chunk(jnp.minimum(c0 + 2, NCH - 1), csets[0])
        process_chunk(c0 + 1, csets[1])
        return carry

    lax.fori_loop(0, NCH // 2, pairc, 0)
    pltpu.make_async_copy(src_hbm.at[pl.ds(0, CE)], esrc0, seme0).wait()
    pltpu.make_async_copy(dst_hbm.at[pl.ds(0, CE)], edst0, seme0).wait()
    pltpu.make_async_copy(jac_hbm.at[pl.ds(0, CE)], ejac0, seme0).wait()

    def flush(r, carry):
        for k in range(D // 16):
            o = r * D + k * 16
            acc_v[pl.ds(o, 16)] = jnp.maximum(acc_v[pl.ds(o, 16)] + bvs[k], 0.0)
        return carry

    lax.fori_loop(0, RPT, flush, 0)
    pltpu.sync_copy(acc_v, h_hbm.at[pl.ds(base * D, RPT * D)])


def _gcn_layer_sc(y, ns, src_p, dstc_p, jac_p, b, c):
    cvec = jnp.broadcast_to(c.astype(jnp.float32), (16,))
    h = _sc_msg_kernel(y, ns, src_p, dstc_p, jac_p, b, cvec)
    return h.reshape(NP, D)


def kernel(x, edge_index, W1, b1, c1, W2, b2, c2, W_out, b_out):
    loops = jnp.arange(N, dtype=edge_index.dtype)
    src = jnp.concatenate([edge_index[0], loops])
    dst = jnp.concatenate([edge_index[1], loops])
    src_p = jnp.zeros((EP,), jnp.int32).at[:E_REAL].set(src)
    dst_p = jnp.zeros((EP,), jnp.int32).at[:E_REAL].set(dst)

    xp = jnp.zeros((NP, D), jnp.float32).at[:N].set(x)

    dstc_p = jnp.full((EP,), 1 << 20, jnp.int32).at[:E_REAL].set(dst)

    pa_flat, degs = _sc_adj_kernel(src_p, dst_p)
    jac_p = _sc_jac_kernel(pa_flat.reshape(NP, W), degs, src_p, dst_p)

    y1, ns1 = _linear(xp, W1)
    h1 = _gcn_layer_sc(y1, ns1[:, 0], src_p, dstc_p, jac_p, b1, c1)
    y2, ns2 = _linear(h1, W2)
    h2 = _gcn_layer_sc(y2, ns2[:, 0], src_p, dstc_p, jac_p, b2, c2)
    return _head(h2, W_out, b_out)
